# Initial kernel scaffold; baseline (speedup 1.0000x reference)
import functools

import jax
import jax.numpy as jnp
from jax import lax
from jax.experimental import pallas as pl
from jax.experimental.pallas import tpu as pltpu
from jax.experimental.pallas import tpu_sc as plsc

N = 10000
D = 128
DP = 144
E = 320000
K = 80
NTILE = 16
NF = DP // 16
BN_SCALE = float(1.0 / jnp.sqrt(jnp.float32(1.0 + 1e-3)))
HIGH = lax.Precision.HIGHEST

_MESH = plsc.VectorSubcoreMesh(core_axis_name="c", subcore_axis_name="s")


def _zero_rows(rows):
    zero = jnp.zeros((16,), jnp.float32)

    @pl.loop(0, K)
    def _(i):
        for f in range(NF):
            rows[i, pl.ds(f * 16, 16)] = zero


def _zero_acc(acc, rows, sid):
    @pl.loop(sid, N // K, step=NTILE)
    def _(t):
        pltpu.sync_copy(rows, acc.at[pl.ds(t * K, K)])


def _dump_acc(acc, out_hbm, sid):
    @pl.loop(sid, N // K, step=NTILE)
    def _(t):
        pltpu.sync_copy(acc.at[pl.ds(t * K, K)], out_hbm.at[pl.ds(t * K, K)])


def _spmm_accumulate(table_hbm, r_hbm, c_hbm, acc, ridx, cidx, vals, rows,
                     e0, nchunks, fill_vals):
    @pl.loop(0, nchunks)
    def _(t):
        base = e0 + t * K
        pltpu.sync_copy(r_hbm.at[pl.ds(base, K)], ridx.at[0])
        pltpu.sync_copy(c_hbm.at[pl.ds(base, K)], cidx.at[0])
        fill_vals(base)
        pltpu.sync_copy(table_hbm.at[cidx.at[0]], rows)

        @pl.loop(0, K)
        def _(i):
            vi = plsc.load_gather(vals, [lax.broadcast(i, (16,))])
            for f in range(NF):
                sl = pl.ds(f * 16, 16)
                rows[i, sl] = rows[i, sl] * vi

        pltpu.sync_copy(rows, acc.at[ridx.at[0]], add=True)


_SC_SCRATCH = [
    pltpu.VMEM_SHARED((N, DP), jnp.float32),
    pltpu.VMEM((1, K), jnp.int32),
    pltpu.VMEM((1, K), jnp.int32),
    pltpu.VMEM((K,), jnp.float32),
    pltpu.VMEM((K, DP), jnp.float32),
]


@functools.partial(
    pl.kernel,
    out_type=(jax.ShapeDtypeStruct((N, DP), jnp.float32),
              jax.ShapeDtypeStruct((N, DP), jnp.float32)),
    mesh=_MESH,
    scratch_types=_SC_SCRATCH + [
        pltpu.VMEM((N,), jnp.float32),
        pltpu.VMEM((N,), jnp.float32),
    ],
)
def _sc1(t1_hbm, ta_hbm, r1_hbm, c1_hbm, a1_hbm, r2_hbm, c2_hbm, a2_hbm,
         s1_hbm, s2_hbm, z1_hbm, z2_hbm,
         acc, ridx, cidx, vals, rows, s1_v, s2_v):
    core = lax.axis_index("c")
    sid = lax.axis_index("s")
    per_tile = E // NTILE
    nchunks = per_tile // K
    e0 = sid * per_tile

    _zero_rows(rows)
    _zero_acc(acc, rows, sid)
    plsc.subcore_barrier()

    @pl.when(core == 0)
    def _():
        def fill_vals(base):
            pltpu.sync_copy(a1_hbm.at[pl.ds(base, K)], vals)

        _spmm_accumulate(t1_hbm, r1_hbm, c1_hbm, acc, ridx, cidx, vals, rows,
                         e0, nchunks, fill_vals)

    @pl.when(core == 1)
    def _():
        pltpu.sync_copy(s1_hbm, s1_v)
        pltpu.sync_copy(s2_hbm, s2_v)

        def fill_vals(base):
            pltpu.sync_copy(a2_hbm.at[pl.ds(base, K)], vals)

            @pl.loop(0, K, step=16)
            def _(j):
                sl = pl.ds(j, 16)
                g1 = plsc.load_gather(s1_v, [ridx[0, sl]])
                g2 = plsc.load_gather(s2_v, [cidx[0, sl]])
                e = vals[sl] * (g1 + g2)
                e = jnp.maximum(e, e * 0.2)
                vals[sl] = jnp.exp(e)

        _spmm_accumulate(ta_hbm, r2_hbm, c2_hbm, acc, ridx, cidx, vals, rows,
                         e0, nchunks, fill_vals)

    plsc.subcore_barrier()

    @pl.when(core == 0)
    def _():
        _dump_acc(acc, z1_hbm, sid)

    @pl.when(core == 1)
    def _():
        _dump_acc(acc, z2_hbm, sid)


@functools.partial(
    pl.kernel,
    out_type=(jax.ShapeDtypeStruct((N, DP), jnp.float32),
              jax.ShapeDtypeStruct((N, DP), jnp.float32)),
    mesh=_MESH,
    scratch_types=_SC_SCRATCH,
)
def _sc2(t2_hbm, r1_hbm, c1_hbm, a1_hbm, za_hbm, zb_hbm,
         acc, ridx, cidx, vals, rows):
    core = lax.axis_index("c")
    sid = lax.axis_index("s")
    half = E // 2
    per_tile = half // NTILE
    nchunks = per_tile // K
    e0 = core * half + sid * per_tile

    _zero_rows(rows)
    _zero_acc(acc, rows, sid)
    plsc.subcore_barrier()

    def fill_vals(base):
        pltpu.sync_copy(a1_hbm.at[pl.ds(base, K)], vals)

    _spmm_accumulate(t2_hbm, r1_hbm, c1_hbm, acc, ridx, cidx, vals, rows,
                     e0, nchunks, fill_vals)

    plsc.subcore_barrier()

    @pl.when(core == 0)
    def _():
        _dump_acc(acc, za_hbm, sid)

    @pl.when(core == 1)
    def _():
        _dump_acc(acc, zb_hbm, sid)


BT = 2000


def _pad_cols(m, width, one_col=False):
    pad = jnp.zeros((m.shape[0], width - m.shape[1]), jnp.float32)
    if one_col:
        pad = jnp.where(lax.broadcasted_iota(jnp.int32, pad.shape, 1) == 0,
                        1.0, 0.0)
    return jnp.concatenate([m, pad], axis=1)


def _tc1_body(x_ref, w1_ref, wa_ref, m1_ref, m2_ref,
              t1_ref, ta_ref, s1_ref, s2_ref):
    xb = x_ref[...] * BN_SCALE
    t1_ref[...] = _pad_cols(jnp.dot(xb, w1_ref[...], precision=HIGH), DP)
    ta_ref[...] = _pad_cols(jnp.dot(xb, wa_ref[...], precision=HIGH), DP,
                            one_col=True)
    s1_ref[...] = jnp.tanh(jnp.sum(
        jnp.dot(xb, m1_ref[...], precision=HIGH) * xb, axis=1, keepdims=True))
    s2_ref[...] = jnp.tanh(jnp.sum(
        jnp.dot(xb, m2_ref[...], precision=HIGH) * xb, axis=1, keepdims=True))


def _tc2_body(z1_ref, z2_ref, b1_ref, wh_ref, w2_ref, y3_ref, t2_ref):
    y1 = jnp.tanh(z1_ref[:, :D] + b1_ref[...])
    den = z2_ref[:, D:D + 1]
    y2 = jnp.where(den != 0.0, z2_ref[:, :D] / den, 0.0)
    i1 = y2 * BN_SCALE
    i2 = y1 * BN_SCALE
    gate = jnp.maximum(jnp.tanh(jnp.dot(i1, wh_ref[...], precision=HIGH)), 0.0)
    y3 = jnp.tanh(i2 * (1.0 - gate) + i1 * gate)
    y3_ref[...] = y3
    t2_ref[...] = _pad_cols(
        jnp.dot(y3 * BN_SCALE, w2_ref[...], precision=HIGH), DP)


def _l2n(v):
    return v * lax.rsqrt(jnp.maximum(jnp.sum(v * v, axis=1, keepdims=True),
                                     1e-12))


def _tc3_body(za_ref, zb_ref, b2_ref, y3_ref, x_ref, out_ref):
    y4 = jnp.tanh(za_ref[:, :D] + zb_ref[:, :D] + b2_ref[...])
    y = jnp.concatenate([_l2n(y3_ref[...]), _l2n(y4), _l2n(x_ref[...])],
                        axis=1)
    out_ref[...] = _l2n(y)


def _row_spec(width):
    return pl.BlockSpec((BT, width), lambda i: (i, 0))


def _full_spec(shape):
    return pl.BlockSpec(shape, lambda i: (0, 0))


_tc1 = pl.pallas_call(
    _tc1_body,
    grid=(N // BT,),
    in_specs=[_row_spec(D)] + [_full_spec((D, D))] * 4,
    out_specs=[_row_spec(DP), _row_spec(DP), _row_spec(1), _row_spec(1)],
    out_shape=(jax.ShapeDtypeStruct((N, DP), jnp.float32),
               jax.ShapeDtypeStruct((N, DP), jnp.float32),
               jax.ShapeDtypeStruct((N, 1), jnp.float32),
               jax.ShapeDtypeStruct((N, 1), jnp.float32)),
)

_tc2 = pl.pallas_call(
    _tc2_body,
    grid=(N // BT,),
    in_specs=[_row_spec(DP), _row_spec(DP), _full_spec((1, D)),
              _full_spec((D, D)), _full_spec((D, D))],
    out_specs=[_row_spec(D), _row_spec(DP)],
    out_shape=(jax.ShapeDtypeStruct((N, D), jnp.float32),
               jax.ShapeDtypeStruct((N, DP), jnp.float32)),
)

_tc3 = pl.pallas_call(
    _tc3_body,
    grid=(N // BT,),
    in_specs=[_row_spec(DP), _row_spec(DP), _full_spec((1, D)),
              _row_spec(D), _row_spec(D)],
    out_specs=_row_spec(3 * D),
    out_shape=jax.ShapeDtypeStruct((N, 3 * D), jnp.float32),
)


def kernel(init_embeds, W1, b1, Wa, M1, M2, Wh, W2, b2,
           a1_values, a2_values, edge_index1, edge_index2):
    x = init_embeds
    r1, c1 = edge_index1[0], edge_index1[1]
    r2, c2 = edge_index2[0], edge_index2[1]

    t1x, tax, s1, s2 = _tc1(x, W1, Wa, M1, M2)
    z1x, z2x = _sc1(t1x, tax, r1, c1, a1_values, r2, c2, a2_values,
                    s1.reshape(N), s2.reshape(N))
    y3, t2x = _tc2(z1x, z2x, b1.reshape(1, D), Wh, W2)
    za, zb = _sc2(t2x, r1, c1, a1_values)
    return _tc3(za, zb, b2.reshape(1, D), y3, x)


# speedup vs baseline: 8.1341x; 8.1341x over previous
import dataclasses
import functools
import math

import jax
import jax.numpy as jnp
from jax import lax
from jax.experimental import pallas as pl
from jax.experimental.pallas import tpu as pltpu
from jax.experimental.pallas import tpu_sc as plsc

N = 10000
NP = 10240
D = 128
E = 320000
K = 80
NTILE = 16
NF = D // 16
BN_SCALE = 1.0 / math.sqrt(1.0 + 1e-3)
HIGH = lax.Precision.HIGHEST

_MESH = plsc.VectorSubcoreMesh(core_axis_name="c", subcore_axis_name="s")
_CP = pltpu.CompilerParams()
if "needs_layout_passes" in pltpu.CompilerParams.__dataclass_fields__:
    _CP = dataclasses.replace(_CP, needs_layout_passes=False)


def _zero_rows(rows):
    zero = jnp.zeros((16,), jnp.float32)

    @pl.loop(0, K)
    def _(i):
        for f in range(NF):
            rows[i, pl.ds(f * 16, 16)] = zero


def _zero_acc(acc, rows, sid):
    @pl.loop(sid, NP // K, step=NTILE)
    def _(t):
        pltpu.sync_copy(rows, acc.at[pl.ds(t * K, K)])


def _dump_acc(acc, out_hbm, sid):
    @pl.loop(sid, NP // K, step=NTILE)
    def _(t):
        pltpu.sync_copy(acc.at[pl.ds(t * K, K)], out_hbm.at[pl.ds(t * K, K)])


def _spmm_accumulate(table_hbm, r_hbm, c_hbm, acc, ridx, cidx, vals, rows,
                     e0, nchunks, fill_vals):
    @pl.loop(0, nchunks)
    def _(t):
        base = e0 + t * K
        pltpu.sync_copy(r_hbm.at[pl.ds(base, K)], ridx.at[0])
        pltpu.sync_copy(c_hbm.at[pl.ds(base, K)], cidx.at[0])
        fill_vals(base)
        pltpu.sync_copy(table_hbm.at[cidx.at[0]], rows)

        @pl.loop(0, K)
        def _(i):
            vi = plsc.load_gather(vals, [lax.broadcast(i, (16,))])
            for f in range(NF):
                sl = pl.ds(f * 16, 16)
                rows[i, sl] = rows[i, sl] * vi

        pltpu.sync_copy(rows, acc.at[ridx.at[0]], add=True)


_SC_SCRATCH = [
    pltpu.VMEM_SHARED((NP, D), jnp.float32),
    pltpu.VMEM((1, K), jnp.int32),
    pltpu.VMEM((1, K), jnp.int32),
    pltpu.VMEM((K,), jnp.float32),
    pltpu.VMEM((K, D), jnp.float32),
]


@functools.partial(
    pl.kernel,
    out_type=(jax.ShapeDtypeStruct((NP, D), jnp.float32),
              jax.ShapeDtypeStruct((NP, D), jnp.float32)),
    mesh=_MESH,
    compiler_params=_CP,
    scratch_types=_SC_SCRATCH + [
        pltpu.VMEM_SHARED((NP // D, D), jnp.float32),
        pltpu.VMEM((N,), jnp.float32),
        pltpu.VMEM((N,), jnp.float32),
        pltpu.VMEM((NP // D, D), jnp.float32),
        pltpu.VMEM((1, NP // D), jnp.int32),
        pltpu.VMEM((32, D), jnp.float32),
        pltpu.VMEM((1, D), jnp.float32),
        pltpu.VMEM((32,), jnp.float32),
    ],
)
def _sc1(t1_hbm, ta_hbm, r1_hbm, c1_hbm, a1_hbm, r2_hbm, c2_hbm, a2_hbm,
         s1_hbm, s2_hbm, z1_hbm, z2_hbm,
         acc, ridx, cidx, vals, rows, den_acc, s1_v, s2_v, den_v, idn,
         dbuf, drow, dscale):
    core = lax.axis_index("c")
    sid = lax.axis_index("s")
    per_tile = E // NTILE
    nchunks = per_tile // K
    e0 = sid * per_tile
    zero = jnp.zeros((16,), jnp.float32)
    lane = lax.iota(jnp.int32, 16)

    _zero_rows(rows)
    _zero_acc(acc, rows, sid)

    @pl.when(jnp.logical_and(core == 1, sid == 0))
    def _():
        pltpu.sync_copy(rows, den_acc)

    plsc.subcore_barrier()

    @pl.when(core == 0)
    def _():
        def fill_vals(base):
            pltpu.sync_copy(a1_hbm.at[pl.ds(base, K)], vals)

        _spmm_accumulate(t1_hbm, r1_hbm, c1_hbm, acc, ridx, cidx, vals, rows,
                         e0, nchunks, fill_vals)

    @pl.when(core == 1)
    def _():
        pltpu.sync_copy(s1_hbm, s1_v)
        pltpu.sync_copy(s2_hbm, s2_v)

        @pl.loop(0, NP // D)
        def _(i):
            for f in range(NF):
                den_v[i, pl.ds(f * 16, 16)] = zero

        @pl.loop(0, NP // D, step=16)
        def _(j):
            idn[0, pl.ds(j, 16)] = lane + j

        masks = [lane == q for q in range(16)]

        def fill_vals(base):
            pltpu.sync_copy(a2_hbm.at[pl.ds(base, K)], vals)

            @pl.loop(0, K, step=16)
            def _(j):
                sl = pl.ds(j, 16)
                ri = ridx[0, sl]
                g1 = plsc.load_gather(s1_v, [ri])
                g2 = plsc.load_gather(s2_v, [cidx[0, sl]])
                e = vals[sl] * (g1 + g2)
                e = jnp.maximum(e, e * 0.2)
                ee = jnp.exp(e)
                vals[sl] = ee
                rhi = lax.shift_right_logical(ri, 7)
                rlo = ri & (D - 1)
                for q in range(16):
                    plsc.addupdate_scatter(den_v, [rhi, rlo], ee,
                                           mask=masks[q])

        _spmm_accumulate(ta_hbm, r2_hbm, c2_hbm, acc, ridx, cidx, vals, rows,
                         e0, nchunks, fill_vals)

        pltpu.sync_copy(den_v, den_acc.at[idn.at[0]], add=True)

    plsc.subcore_barrier()

    @pl.when(core == 0)
    def _():
        _dump_acc(acc, z1_hbm, sid)

    @pl.when(core == 1)
    def _():
        @pl.loop(sid, NP // 32, step=NTILE)
        def _(m):
            pltpu.sync_copy(acc.at[pl.ds(m * 32, 32)], dbuf)
            pltpu.sync_copy(den_acc.at[pl.ds(m // 4, 1)], drow)
            q = (m % 4) * 32
            for f in range(2):
                dv = drow[0, pl.ds(q + f * 16, 16)]
                dscale[pl.ds(f * 16, 16)] = jnp.where(dv > 0.0, 1.0 / dv, 0.0)

            @pl.loop(0, 32)
            def _(i):
                si = plsc.load_gather(dscale, [lax.broadcast(i, (16,))])
                for f in range(NF):
                    sl = pl.ds(f * 16, 16)
                    dbuf[i, sl] = dbuf[i, sl] * si

            pltpu.sync_copy(dbuf, z2_hbm.at[pl.ds(m * 32, 32)])


@functools.partial(
    pl.kernel,
    out_type=(jax.ShapeDtypeStruct((NP, D), jnp.float32),
              jax.ShapeDtypeStruct((NP, D), jnp.float32)),
    mesh=_MESH,
    compiler_params=_CP,
    scratch_types=_SC_SCRATCH,
)
def _sc2(t2_hbm, r1_hbm, c1_hbm, a1_hbm, za_hbm, zb_hbm,
         acc, ridx, cidx, vals, rows):
    core = lax.axis_index("c")
    sid = lax.axis_index("s")
    half = E // 2
    per_tile = half // NTILE
    nchunks = per_tile // K
    e0 = core * half + sid * per_tile

    _zero_rows(rows)
    _zero_acc(acc, rows, sid)
    plsc.subcore_barrier()

    def fill_vals(base):
        pltpu.sync_copy(a1_hbm.at[pl.ds(base, K)], vals)

    _spmm_accumulate(t2_hbm, r1_hbm, c1_hbm, acc, ridx, cidx, vals, rows,
                     e0, nchunks, fill_vals)

    plsc.subcore_barrier()

    @pl.when(core == 0)
    def _():
        _dump_acc(acc, za_hbm, sid)

    @pl.when(core == 1)
    def _():
        _dump_acc(acc, zb_hbm, sid)


BT = 2000


def _tc1_body(x_ref, w1_ref, wa_ref, m1_ref, m2_ref,
              t1_ref, ta_ref, s1_ref, s2_ref):
    xb = x_ref[...] * BN_SCALE
    t1_ref[...] = jnp.dot(xb, w1_ref[...], precision=HIGH)
    ta_ref[...] = jnp.dot(xb, wa_ref[...], precision=HIGH)
    s1_ref[...] = jnp.tanh(jnp.sum(
        jnp.dot(xb, m1_ref[...], precision=HIGH) * xb, axis=1, keepdims=True))
    s2_ref[...] = jnp.tanh(jnp.sum(
        jnp.dot(xb, m2_ref[...], precision=HIGH) * xb, axis=1, keepdims=True))


def _tc2_body(z1_ref, y2_ref, b1_ref, wh_ref, w2_ref, y3_ref, t2_ref):
    y1 = jnp.tanh(z1_ref[...] + b1_ref[...])
    i1 = y2_ref[...] * BN_SCALE
    i2 = y1 * BN_SCALE
    gate = jnp.maximum(jnp.tanh(jnp.dot(i1, wh_ref[...], precision=HIGH)), 0.0)
    y3 = jnp.tanh(i2 * (1.0 - gate) + i1 * gate)
    y3_ref[...] = y3
    t2_ref[...] = jnp.dot(y3 * BN_SCALE, w2_ref[...], precision=HIGH)


def _l2n(v):
    return v * lax.rsqrt(jnp.maximum(jnp.sum(v * v, axis=1, keepdims=True),
                                     1e-12))


def _tc3_body(za_ref, zb_ref, b2_ref, y3_ref, x_ref, out_ref):
    y4 = jnp.tanh(za_ref[...] + zb_ref[...] + b2_ref[...])
    y = jnp.concatenate([_l2n(y3_ref[...]), _l2n(y4), _l2n(x_ref[...])],
                        axis=1)
    out_ref[...] = _l2n(y)


def _row_spec(width):
    return pl.BlockSpec((BT, width), lambda i: (i, 0))


def _full_spec(shape):
    return pl.BlockSpec(shape, lambda i: tuple(0 for _ in shape))


_tc1 = pl.pallas_call(
    _tc1_body,
    grid=(N // BT,),
    in_specs=[_row_spec(D)] + [_full_spec((D, D))] * 4,
    out_specs=[_row_spec(D), _row_spec(D), _row_spec(1), _row_spec(1)],
    out_shape=(jax.ShapeDtypeStruct((N, D), jnp.float32),
               jax.ShapeDtypeStruct((N, D), jnp.float32),
               jax.ShapeDtypeStruct((N, 1), jnp.float32),
               jax.ShapeDtypeStruct((N, 1), jnp.float32)),
)

_tc2 = pl.pallas_call(
    _tc2_body,
    grid=(N // BT,),
    in_specs=[_row_spec(D), _row_spec(D), _full_spec((1, D)),
              _full_spec((D, D)), _full_spec((D, D))],
    out_specs=[_row_spec(D), _row_spec(D)],
    out_shape=(jax.ShapeDtypeStruct((N, D), jnp.float32),
               jax.ShapeDtypeStruct((N, D), jnp.float32)),
)

_tc3 = pl.pallas_call(
    _tc3_body,
    grid=(N // BT,),
    in_specs=[_row_spec(D), _row_spec(D), _full_spec((1, D)),
              _row_spec(D), _row_spec(D)],
    out_specs=_row_spec(3 * D),
    out_shape=jax.ShapeDtypeStruct((N, 3 * D), jnp.float32),
)


def kernel(init_embeds, W1, b1, Wa, M1, M2, Wh, W2, b2,
           a1_values, a2_values, edge_index1, edge_index2):
    x = init_embeds
    r1, c1 = edge_index1[0], edge_index1[1]
    r2, c2 = edge_index2[0], edge_index2[1]

    t1x, tax, s1, s2 = _tc1(x, W1, Wa, M1, M2)
    z1, y2 = _sc1(t1x, tax, r1, c1, a1_values, r2, c2, a2_values,
                  s1.reshape(N), s2.reshape(N))
    y3, t2x = _tc2(z1[:N], y2[:N], b1.reshape(1, D), Wh, W2)
    za, zb = _sc2(t2x, r1, c1, a1_values)
    return _tc3(za[:N], zb[:N], b2.reshape(1, D), y3, x)


# speedup vs baseline: 10.8827x; 1.3379x over previous
import dataclasses
import functools
import math

import jax
import jax.numpy as jnp
from jax import lax
from jax.experimental import pallas as pl
from jax.experimental.pallas import tpu as pltpu
from jax.experimental.pallas import tpu_sc as plsc

N = 10000
NP = 10240
D = 128
E = 320000
K = 80
NTILE = 16
NF = D // 16
BN_SCALE = 1.0 / math.sqrt(1.0 + 1e-3)
HIGH = lax.Precision.HIGHEST

_MESH = plsc.VectorSubcoreMesh(core_axis_name="c", subcore_axis_name="s")
_CP = pltpu.CompilerParams()
if "needs_layout_passes" in pltpu.CompilerParams.__dataclass_fields__:
    _CP = dataclasses.replace(_CP, needs_layout_passes=False)


def _zero_rows(rows):
    zero = jnp.zeros((16,), jnp.float32)

    @pl.loop(0, K)
    def _(i):
        for f in range(NF):
            rows[i, pl.ds(f * 16, 16)] = zero


def _zero_acc(acc, rows, sid):
    @pl.loop(sid, NP // K, step=NTILE)
    def _(t):
        pltpu.sync_copy(rows, acc.at[pl.ds(t * K, K)])


def _dump_acc(acc, out_hbm, sid):
    @pl.loop(sid, NP // K, step=NTILE)
    def _(t):
        pltpu.sync_copy(acc.at[pl.ds(t * K, K)], out_hbm.at[pl.ds(t * K, K)])


def _spmm_accumulate(table_hbm, r_hbm, c_hbm, acc, bufa, bufb,
                     e0, nchunks, fill_vals):
    def load_idx(buf, t):
        ridx, cidx, vals, rows, sem = buf
        base = e0 + t * K
        pltpu.sync_copy(r_hbm.at[pl.ds(base, K)], ridx.at[0])
        pltpu.sync_copy(c_hbm.at[pl.ds(base, K)], cidx.at[0])
        pltpu.async_copy(table_hbm.at[cidx.at[0]], rows, sem)
        fill_vals(buf, base)

    def process(buf):
        ridx, cidx, vals, rows, sem = buf
        pltpu.make_async_copy(table_hbm.at[cidx.at[0]], rows, sem).wait()

        @pl.loop(0, K)
        def _(i):
            vi = plsc.load_gather(vals, [lax.broadcast(i, (16,))])
            for f in range(NF):
                sl = pl.ds(f * 16, 16)
                rows[i, sl] = rows[i, sl] * vi

        pltpu.sync_copy(rows, acc.at[ridx.at[0]], add=True)

    load_idx(bufa, 0)

    @pl.loop(0, 2 * ((nchunks - 1) // 2), step=2)
    def _(t):
        load_idx(bufb, t + 1)
        process(bufa)
        load_idx(bufa, t + 2)
        process(bufb)

    if nchunks % 2 == 0:
        load_idx(bufb, nchunks - 1)
        process(bufa)
        process(bufb)
    else:
        process(bufa)


def _edge_buffers():
    return 2 * [
        pltpu.VMEM((1, K), jnp.int32),
        pltpu.VMEM((1, K), jnp.int32),
        pltpu.VMEM((K,), jnp.float32),
        pltpu.VMEM((K, D), jnp.float32),
        pltpu.SemaphoreType.DMA,
    ]


_SC_SCRATCH = [pltpu.VMEM_SHARED((NP, D), jnp.float32)] + _edge_buffers()


@functools.partial(
    pl.kernel,
    out_type=(jax.ShapeDtypeStruct((NP, D), jnp.float32),
              jax.ShapeDtypeStruct((NP, D), jnp.float32)),
    mesh=_MESH,
    compiler_params=_CP,
    scratch_types=_SC_SCRATCH + [
        pltpu.VMEM_SHARED((NP // D, D), jnp.float32),
        pltpu.VMEM((N,), jnp.int32),
        pltpu.VMEM((NP // D, D), jnp.float32),
        pltpu.VMEM((1, NP // D), jnp.int32),
        pltpu.VMEM((32, D), jnp.float32),
        pltpu.VMEM((1, D), jnp.float32),
        pltpu.VMEM((32,), jnp.float32),
    ],
)
def _sc1(t1_hbm, ta_hbm, r1_hbm, c1_hbm, a1_hbm, r2_hbm, c2_hbm, a2_hbm,
         sp_hbm, z1_hbm, z2_hbm,
         acc, ra, ca, va, rwa, sma, rb, cb, vb, rwb, smb,
         den_acc, s_v, den_v, idn, dbuf, drow, dscale):
    core = lax.axis_index("c")
    sid = lax.axis_index("s")
    per_tile = E // NTILE
    nchunks = per_tile // K
    e0 = sid * per_tile
    zero = jnp.zeros((16,), jnp.float32)
    lane = lax.iota(jnp.int32, 16)
    bufa = (ra, ca, va, rwa, sma)
    bufb = (rb, cb, vb, rwb, smb)

    _zero_rows(rwa)
    _zero_acc(acc, rwa, sid)

    @pl.when(jnp.logical_and(core == 1, sid == 0))
    def _():
        pltpu.sync_copy(rwa, den_acc)

    plsc.subcore_barrier()

    @pl.when(core == 0)
    def _():
        def fill_vals(buf, base):
            pltpu.sync_copy(a1_hbm.at[pl.ds(base, K)], buf[2])

        _spmm_accumulate(t1_hbm, r1_hbm, c1_hbm, acc, bufa, bufb,
                         e0, nchunks, fill_vals)

    @pl.when(core == 1)
    def _():
        pltpu.sync_copy(sp_hbm, s_v)

        @pl.loop(0, NP // D)
        def _(i):
            for f in range(NF):
                den_v[i, pl.ds(f * 16, 16)] = zero

        @pl.loop(0, NP // D, step=16)
        def _(j):
            idn[0, pl.ds(j, 16)] = lane + j

        masks = [lane == q for q in range(16)]

        def fill_vals(buf, base):
            ridx, cidx, vals = buf[0], buf[1], buf[2]
            pltpu.sync_copy(a2_hbm.at[pl.ds(base, K)], vals)

            @pl.loop(0, K, step=16)
            def _(j):
                sl = pl.ds(j, 16)
                ri = ridx[0, sl]
                p1 = plsc.load_gather(s_v, [ri])
                p2 = plsc.load_gather(s_v, [cidx[0, sl]])
                g1 = lax.shift_right_logical(p1, 16).astype(jnp.float32)
                g2 = (p2 & 0xFFFF).astype(jnp.float32)
                e = vals[sl] * ((g1 + g2 - 65536.0) * (1.0 / 32767.0))
                e = jnp.maximum(e, e * 0.2)
                ee = jnp.exp(e)
                vals[sl] = ee
                rhi = lax.shift_right_logical(ri, 7)
                rlo = ri & (D - 1)
                for q in range(16):
                    plsc.addupdate_scatter(den_v, [rhi, rlo], ee,
                                           mask=masks[q])

        _spmm_accumulate(ta_hbm, r2_hbm, c2_hbm, acc, bufa, bufb,
                         e0, nchunks, fill_vals)

        pltpu.sync_copy(den_v, den_acc.at[idn.at[0]], add=True)

    plsc.subcore_barrier()

    @pl.when(core == 0)
    def _():
        _dump_acc(acc, z1_hbm, sid)

    @pl.when(core == 1)
    def _():
        @pl.loop(sid, NP // 32, step=NTILE)
        def _(m):
            pltpu.sync_copy(acc.at[pl.ds(m * 32, 32)], dbuf)
            pltpu.sync_copy(den_acc.at[pl.ds(m // 4, 1)], drow)
            q = (m % 4) * 32
            for f in range(2):
                dv = drow[0, pl.ds(q + f * 16, 16)]
                dscale[pl.ds(f * 16, 16)] = jnp.where(dv > 0.0, 1.0 / dv, 0.0)

            @pl.loop(0, 32)
            def _(i):
                si = plsc.load_gather(dscale, [lax.broadcast(i, (16,))])
                for f in range(NF):
                    sl = pl.ds(f * 16, 16)
                    dbuf[i, sl] = dbuf[i, sl] * si

            pltpu.sync_copy(dbuf, z2_hbm.at[pl.ds(m * 32, 32)])


@functools.partial(
    pl.kernel,
    out_type=(jax.ShapeDtypeStruct((NP, D), jnp.float32),
              jax.ShapeDtypeStruct((NP, D), jnp.float32)),
    mesh=_MESH,
    compiler_params=_CP,
    scratch_types=_SC_SCRATCH,
)
def _sc2(t2_hbm, r1_hbm, c1_hbm, a1_hbm, za_hbm, zb_hbm,
         acc, ra, ca, va, rwa, sma, rb, cb, vb, rwb, smb):
    core = lax.axis_index("c")
    sid = lax.axis_index("s")
    half = E // 2
    per_tile = half // NTILE
    nchunks = per_tile // K
    e0 = core * half + sid * per_tile

    _zero_rows(rwa)
    _zero_acc(acc, rwa, sid)
    plsc.subcore_barrier()

    def fill_vals(buf, base):
        pltpu.sync_copy(a1_hbm.at[pl.ds(base, K)], buf[2])

    _spmm_accumulate(t2_hbm, r1_hbm, c1_hbm, acc, (ra, ca, va, rwa, sma),
                     (rb, cb, vb, rwb, smb), e0, nchunks, fill_vals)

    plsc.subcore_barrier()

    @pl.when(core == 0)
    def _():
        _dump_acc(acc, za_hbm, sid)

    @pl.when(core == 1)
    def _():
        _dump_acc(acc, zb_hbm, sid)


BT = 2000


def _tc1_body(x_ref, w1_ref, wa_ref, m1_ref, m2_ref,
              t1_ref, ta_ref, sp_ref):
    xb = x_ref[...] * BN_SCALE
    t1_ref[...] = jnp.dot(xb, w1_ref[...], precision=HIGH)
    ta_ref[...] = jnp.dot(xb, wa_ref[...], precision=HIGH)
    s1 = jnp.tanh(jnp.sum(
        jnp.dot(xb, m1_ref[...], precision=HIGH) * xb, axis=1, keepdims=True))
    s2 = jnp.tanh(jnp.sum(
        jnp.dot(xb, m2_ref[...], precision=HIGH) * xb, axis=1, keepdims=True))
    q1 = jnp.round(s1 * 32767.0).astype(jnp.int32) + 32768
    q2 = jnp.round(s2 * 32767.0).astype(jnp.int32) + 32768
    sp_ref[...] = lax.shift_left(q1, 16) | q2


def _tc2_body(z1_ref, y2_ref, b1_ref, wh_ref, w2_ref, y3_ref, t2_ref):
    y1 = jnp.tanh(z1_ref[...] + b1_ref[...])
    i1 = y2_ref[...] * BN_SCALE
    i2 = y1 * BN_SCALE
    gate = jnp.maximum(jnp.tanh(jnp.dot(i1, wh_ref[...], precision=HIGH)), 0.0)
    y3 = jnp.tanh(i2 * (1.0 - gate) + i1 * gate)
    y3_ref[...] = y3
    t2_ref[...] = jnp.dot(y3 * BN_SCALE, w2_ref[...], precision=HIGH)


def _l2n(v):
    return v * lax.rsqrt(jnp.maximum(jnp.sum(v * v, axis=1, keepdims=True),
                                     1e-12))


def _tc3_body(za_ref, zb_ref, b2_ref, y3_ref, x_ref, out_ref):
    y4 = jnp.tanh(za_ref[...] + zb_ref[...] + b2_ref[...])
    y = jnp.concatenate([_l2n(y3_ref[...]), _l2n(y4), _l2n(x_ref[...])],
                        axis=1)
    out_ref[...] = _l2n(y)


def _row_spec(width):
    return pl.BlockSpec((BT, width), lambda i: (i, 0))


def _full_spec(shape):
    return pl.BlockSpec(shape, lambda i: tuple(0 for _ in shape))


_tc1 = pl.pallas_call(
    _tc1_body,
    grid=(N // BT,),
    in_specs=[_row_spec(D)] + [_full_spec((D, D))] * 4,
    out_specs=[_row_spec(D), _row_spec(D), _row_spec(1)],
    out_shape=(jax.ShapeDtypeStruct((N, D), jnp.float32),
               jax.ShapeDtypeStruct((N, D), jnp.float32),
               jax.ShapeDtypeStruct((N, 1), jnp.int32)),
)

_tc2 = pl.pallas_call(
    _tc2_body,
    grid=(N // BT,),
    in_specs=[_row_spec(D), _row_spec(D), _full_spec((1, D)),
              _full_spec((D, D)), _full_spec((D, D))],
    out_specs=[_row_spec(D), _row_spec(D)],
    out_shape=(jax.ShapeDtypeStruct((N, D), jnp.float32),
               jax.ShapeDtypeStruct((N, D), jnp.float32)),
)

_tc3 = pl.pallas_call(
    _tc3_body,
    grid=(N // BT,),
    in_specs=[_row_spec(D), _row_spec(D), _full_spec((1, D)),
              _row_spec(D), _row_spec(D)],
    out_specs=_row_spec(3 * D),
    out_shape=jax.ShapeDtypeStruct((N, 3 * D), jnp.float32),
)


def kernel(init_embeds, W1, b1, Wa, M1, M2, Wh, W2, b2,
           a1_values, a2_values, edge_index1, edge_index2):
    x = init_embeds
    r1, c1 = edge_index1[0], edge_index1[1]
    r2, c2 = edge_index2[0], edge_index2[1]

    t1x, tax, sp = _tc1(x, W1, Wa, M1, M2)
    z1, y2 = _sc1(t1x, tax, r1, c1, a1_values, r2, c2, a2_values,
                  sp.reshape(N))
    y3, t2x = _tc2(z1[:N], y2[:N], b1.reshape(1, D), Wh, W2)
    za, zb = _sc2(t2x, r1, c1, a1_values)
    return _tc3(za[:N], zb[:N], b2.reshape(1, D), y3, x)


# speedup vs baseline: 12.4952x; 1.1482x over previous
import dataclasses
import functools
import math

import jax
import jax.numpy as jnp
from jax import lax
from jax.experimental import pallas as pl
from jax.experimental.pallas import tpu as pltpu
from jax.experimental.pallas import tpu_sc as plsc

N = 10000
NP = 10240
D = 128
E = 320000
K = 80
NTILE = 16
NF = D // 16
BN_SCALE = 1.0 / math.sqrt(1.0 + 1e-3)
HIGH = lax.Precision.HIGHEST

_MESH = plsc.VectorSubcoreMesh(core_axis_name="c", subcore_axis_name="s")
_CP = pltpu.CompilerParams()
if "needs_layout_passes" in pltpu.CompilerParams.__dataclass_fields__:
    _CP = dataclasses.replace(_CP, needs_layout_passes=False)


def _zero_rows(rows):
    zero = jnp.zeros((16,), jnp.float32)

    @pl.loop(0, K)
    def _(i):
        for f in range(NF):
            rows[i, pl.ds(f * 16, 16)] = zero


def _zero_acc(acc, rows, sid):
    @pl.loop(sid, NP // K, step=NTILE)
    def _(t):
        pltpu.sync_copy(rows, acc.at[pl.ds(t * K, K)])


def _dump_acc(acc, out_hbm, sid):
    @pl.loop(sid, NP // K, step=NTILE)
    def _(t):
        pltpu.sync_copy(acc.at[pl.ds(t * K, K)], out_hbm.at[pl.ds(t * K, K)])


def _spmm_accumulate(table_hbm, r_hbm, c_hbm, acc, bufa, bufb,
                     e0, nchunks, fill_vals):
    def load_idx(buf, t):
        ridx, cidx, vals, rows, sem = buf
        base = e0 + t * K
        pltpu.sync_copy(r_hbm.at[pl.ds(base, K)], ridx.at[0])
        pltpu.sync_copy(c_hbm.at[pl.ds(base, K)], cidx.at[0])
        pltpu.async_copy(table_hbm.at[cidx.at[0]], rows, sem)
        fill_vals(buf, base)

    def process(buf):
        ridx, cidx, vals, rows, sem = buf
        pltpu.make_async_copy(table_hbm.at[cidx.at[0]], rows, sem).wait()

        @plsc.parallel_loop(0, K, unroll=4)
        def _(i):
            vi = plsc.load_gather(vals, [lax.broadcast(i, (16,))])
            for f in range(NF):
                sl = pl.ds(f * 16, 16)
                rows[i, sl] = rows[i, sl] * vi

        pltpu.sync_copy(rows, acc.at[ridx.at[0]], add=True)

    load_idx(bufa, 0)

    @pl.loop(0, 2 * ((nchunks - 1) // 2), step=2)
    def _(t):
        load_idx(bufb, t + 1)
        process(bufa)
        load_idx(bufa, t + 2)
        process(bufb)

    if nchunks % 2 == 0:
        load_idx(bufb, nchunks - 1)
        process(bufa)
        process(bufb)
    else:
        process(bufa)


def _edge_buffers():
    return 2 * [
        pltpu.VMEM((1, K), jnp.int32),
        pltpu.VMEM((1, K), jnp.int32),
        pltpu.VMEM((K,), jnp.float32),
        pltpu.VMEM((K, D), jnp.float32),
        pltpu.SemaphoreType.DMA,
    ]


_SC_SCRATCH = [pltpu.VMEM_SHARED((NP, D), jnp.float32)] + _edge_buffers()


@functools.partial(
    pl.kernel,
    out_type=(jax.ShapeDtypeStruct((NP, D), jnp.float32),
              jax.ShapeDtypeStruct((NP, D), jnp.float32)),
    mesh=_MESH,
    compiler_params=_CP,
    scratch_types=_SC_SCRATCH + [
        pltpu.VMEM_SHARED((NP // D, D), jnp.float32),
        pltpu.VMEM((N,), jnp.int32),
        pltpu.VMEM((NP // D, D), jnp.float32),
        pltpu.VMEM((1, NP // D), jnp.int32),
        pltpu.VMEM((32, D), jnp.float32),
        pltpu.VMEM((1, D), jnp.float32),
        pltpu.VMEM((32,), jnp.float32),
    ],
)
def _sc1(t1_hbm, ta_hbm, r1_hbm, c1_hbm, a1_hbm, r2_hbm, c2_hbm, a2_hbm,
         sp_hbm, z1_hbm, z2_hbm,
         acc, ra, ca, va, rwa, sma, rb, cb, vb, rwb, smb,
         den_acc, s_v, den_v, idn, dbuf, drow, dscale):
    core = lax.axis_index("c")
    sid = lax.axis_index("s")
    per_tile = E // NTILE
    nchunks = per_tile // K
    e0 = sid * per_tile
    zero = jnp.zeros((16,), jnp.float32)
    lane = lax.iota(jnp.int32, 16)
    bufa = (ra, ca, va, rwa, sma)
    bufb = (rb, cb, vb, rwb, smb)

    _zero_rows(rwa)
    _zero_acc(acc, rwa, sid)

    @pl.when(jnp.logical_and(core == 1, sid == 0))
    def _():
        pltpu.sync_copy(rwa, den_acc)

    plsc.subcore_barrier()

    @pl.when(core == 0)
    def _():
        def fill_vals(buf, base):
            pltpu.sync_copy(a1_hbm.at[pl.ds(base, K)], buf[2])

        _spmm_accumulate(t1_hbm, r1_hbm, c1_hbm, acc, bufa, bufb,
                         e0, nchunks, fill_vals)

    @pl.when(core == 1)
    def _():
        pltpu.sync_copy(sp_hbm, s_v)

        @pl.loop(0, NP // D)
        def _(i):
            for f in range(NF):
                den_v[i, pl.ds(f * 16, 16)] = zero

        @pl.loop(0, NP // D, step=16)
        def _(j):
            idn[0, pl.ds(j, 16)] = lane + j

        masks = [lane == q for q in range(16)]

        def fill_vals(buf, base):
            ridx, cidx, vals = buf[0], buf[1], buf[2]
            pltpu.sync_copy(a2_hbm.at[pl.ds(base, K)], vals)

            @pl.loop(0, K, step=16)
            def _(j):
                sl = pl.ds(j, 16)
                ri = ridx[0, sl]
                p1 = plsc.load_gather(s_v, [ri])
                p2 = plsc.load_gather(s_v, [cidx[0, sl]])
                g1 = lax.shift_right_logical(p1, 16).astype(jnp.float32)
                g2 = (p2 & 0xFFFF).astype(jnp.float32)
                e = vals[sl] * ((g1 + g2 - 65536.0) * (1.0 / 32767.0))
                e = jnp.maximum(e, e * 0.2)
                ee = jnp.exp(e)
                vals[sl] = ee
                rhi = lax.shift_right_logical(ri, 7)
                rlo = ri & (D - 1)
                for q in range(16):
                    plsc.addupdate_scatter(den_v, [rhi, rlo], ee,
                                           mask=masks[q])

        _spmm_accumulate(ta_hbm, r2_hbm, c2_hbm, acc, bufa, bufb,
                         e0, nchunks, fill_vals)

        pltpu.sync_copy(den_v, den_acc.at[idn.at[0]], add=True)

    plsc.subcore_barrier()

    @pl.when(core == 0)
    def _():
        _dump_acc(acc, z1_hbm, sid)

    @pl.when(core == 1)
    def _():
        @pl.loop(sid, NP // 32, step=NTILE)
        def _(m):
            pltpu.sync_copy(acc.at[pl.ds(m * 32, 32)], dbuf)
            pltpu.sync_copy(den_acc.at[pl.ds(m // 4, 1)], drow)
            q = (m % 4) * 32
            for f in range(2):
                dv = drow[0, pl.ds(q + f * 16, 16)]
                dscale[pl.ds(f * 16, 16)] = jnp.where(dv > 0.0, 1.0 / dv, 0.0)

            @pl.loop(0, 32)
            def _(i):
                si = plsc.load_gather(dscale, [lax.broadcast(i, (16,))])
                for f in range(NF):
                    sl = pl.ds(f * 16, 16)
                    dbuf[i, sl] = dbuf[i, sl] * si

            pltpu.sync_copy(dbuf, z2_hbm.at[pl.ds(m * 32, 32)])


@functools.partial(
    pl.kernel,
    out_type=(jax.ShapeDtypeStruct((NP, D), jnp.float32),
              jax.ShapeDtypeStruct((NP, D), jnp.float32)),
    mesh=_MESH,
    compiler_params=_CP,
    scratch_types=_SC_SCRATCH,
)
def _sc2(t2_hbm, r1_hbm, c1_hbm, a1_hbm, za_hbm, zb_hbm,
         acc, ra, ca, va, rwa, sma, rb, cb, vb, rwb, smb):
    core = lax.axis_index("c")
    sid = lax.axis_index("s")
    half = E // 2
    per_tile = half // NTILE
    nchunks = per_tile // K
    e0 = core * half + sid * per_tile

    _zero_rows(rwa)
    _zero_acc(acc, rwa, sid)
    plsc.subcore_barrier()

    def fill_vals(buf, base):
        pltpu.sync_copy(a1_hbm.at[pl.ds(base, K)], buf[2])

    _spmm_accumulate(t2_hbm, r1_hbm, c1_hbm, acc, (ra, ca, va, rwa, sma),
                     (rb, cb, vb, rwb, smb), e0, nchunks, fill_vals)

    plsc.subcore_barrier()

    @pl.when(core == 0)
    def _():
        _dump_acc(acc, za_hbm, sid)

    @pl.when(core == 1)
    def _():
        _dump_acc(acc, zb_hbm, sid)


BT = 2000


def _tc1_body(x_ref, w1_ref, wa_ref, m1_ref, m2_ref,
              t1_ref, ta_ref, sp_ref):
    xb = x_ref[...] * BN_SCALE
    t1_ref[...] = jnp.dot(xb, w1_ref[...], precision=HIGH)
    ta_ref[...] = jnp.dot(xb, wa_ref[...], precision=HIGH)
    s1 = jnp.tanh(jnp.sum(
        jnp.dot(xb, m1_ref[...], precision=HIGH) * xb, axis=1, keepdims=True))
    s2 = jnp.tanh(jnp.sum(
        jnp.dot(xb, m2_ref[...], precision=HIGH) * xb, axis=1, keepdims=True))
    q1 = jnp.round(s1 * 32767.0).astype(jnp.int32) + 32768
    q2 = jnp.round(s2 * 32767.0).astype(jnp.int32) + 32768
    sp_ref[...] = lax.shift_left(q1, 16) | q2


def _tc2_body(z1_ref, y2_ref, b1_ref, wh_ref, w2_ref, y3_ref, t2_ref):
    y1 = jnp.tanh(z1_ref[...] + b1_ref[...])
    i1 = y2_ref[...] * BN_SCALE
    i2 = y1 * BN_SCALE
    gate = jnp.maximum(jnp.tanh(jnp.dot(i1, wh_ref[...], precision=HIGH)), 0.0)
    y3 = jnp.tanh(i2 * (1.0 - gate) + i1 * gate)
    y3_ref[...] = y3
    t2_ref[...] = jnp.dot(y3 * BN_SCALE, w2_ref[...], precision=HIGH)


def _l2n(v):
    return v * lax.rsqrt(jnp.maximum(jnp.sum(v * v, axis=1, keepdims=True),
                                     1e-12))


def _tc3_body(za_ref, zb_ref, b2_ref, y3_ref, x_ref, out_ref):
    y4 = jnp.tanh(za_ref[...] + zb_ref[...] + b2_ref[...])
    y = jnp.concatenate([_l2n(y3_ref[...]), _l2n(y4), _l2n(x_ref[...])],
                        axis=1)
    out_ref[...] = _l2n(y)


def _row_spec(width):
    return pl.BlockSpec((BT, width), lambda i: (i, 0))


def _full_spec(shape):
    return pl.BlockSpec(shape, lambda i: tuple(0 for _ in shape))


_tc1 = pl.pallas_call(
    _tc1_body,
    grid=(N // BT,),
    in_specs=[_row_spec(D)] + [_full_spec((D, D))] * 4,
    out_specs=[_row_spec(D), _row_spec(D), _row_spec(1)],
    out_shape=(jax.ShapeDtypeStruct((N, D), jnp.float32),
               jax.ShapeDtypeStruct((N, D), jnp.float32),
               jax.ShapeDtypeStruct((N, 1), jnp.int32)),
)

_tc2 = pl.pallas_call(
    _tc2_body,
    grid=(N // BT,),
    in_specs=[_row_spec(D), _row_spec(D), _full_spec((1, D)),
              _full_spec((D, D)), _full_spec((D, D))],
    out_specs=[_row_spec(D), _row_spec(D)],
    out_shape=(jax.ShapeDtypeStruct((N, D), jnp.float32),
               jax.ShapeDtypeStruct((N, D), jnp.float32)),
)

_tc3 = pl.pallas_call(
    _tc3_body,
    grid=(N // BT,),
    in_specs=[_row_spec(D), _row_spec(D), _full_spec((1, D)),
              _row_spec(D), _row_spec(D)],
    out_specs=_row_spec(3 * D),
    out_shape=jax.ShapeDtypeStruct((N, 3 * D), jnp.float32),
)


def kernel(init_embeds, W1, b1, Wa, M1, M2, Wh, W2, b2,
           a1_values, a2_values, edge_index1, edge_index2):
    x = init_embeds
    r1, c1 = edge_index1[0], edge_index1[1]
    r2, c2 = edge_index2[0], edge_index2[1]

    t1x, tax, sp = _tc1(x, W1, Wa, M1, M2)
    z1, y2 = _sc1(t1x, tax, r1, c1, a1_values, r2, c2, a2_values,
                  sp.reshape(N))
    y3, t2x = _tc2(z1[:N], y2[:N], b1.reshape(1, D), Wh, W2)
    za, zb = _sc2(t2x, r1, c1, a1_values)
    return _tc3(za[:N], zb[:N], b2.reshape(1, D), y3, x)


# speedup vs baseline: 12.6533x; 1.0126x over previous
import dataclasses
import functools
import math

import jax
import jax.numpy as jnp
from jax import lax
from jax.experimental import pallas as pl
from jax.experimental.pallas import tpu as pltpu
from jax.experimental.pallas import tpu_sc as plsc

N = 10000
NP = 10240
D = 128
E = 320000
K = 80
NTILE = 16
NF = D // 16
BN_SCALE = 1.0 / math.sqrt(1.0 + 1e-3)
HIGH = lax.Precision.HIGHEST

_MESH = plsc.VectorSubcoreMesh(core_axis_name="c", subcore_axis_name="s")
_CP = pltpu.CompilerParams()
if "needs_layout_passes" in pltpu.CompilerParams.__dataclass_fields__:
    _CP = dataclasses.replace(_CP, needs_layout_passes=False)


def _zero_rows(rows):
    zero = jnp.zeros((16,), jnp.float32)

    @pl.loop(0, K)
    def _(i):
        for f in range(NF):
            rows[i, pl.ds(f * 16, 16)] = zero


def _zero_acc(acc, rows, sid):
    @pl.loop(sid, NP // K, step=NTILE)
    def _(t):
        pltpu.sync_copy(rows, acc.at[pl.ds(t * K, K)])


def _dump_acc(acc, out_hbm, sid):
    @pl.loop(sid, NP // K, step=NTILE)
    def _(t):
        pltpu.sync_copy(acc.at[pl.ds(t * K, K)], out_hbm.at[pl.ds(t * K, K)])


def _spmm_accumulate(table_hbm, r_hbm, c_hbm, acc, bufa, bufb,
                     e0, nchunks, fill_vals):
    def load_idx(buf, t):
        ridx, cidx, vals, rows, sem = buf
        base = e0 + t * K
        pltpu.sync_copy(r_hbm.at[pl.ds(base, K)], ridx.at[0])
        pltpu.sync_copy(c_hbm.at[pl.ds(base, K)], cidx.at[0])
        pltpu.async_copy(table_hbm.at[cidx.at[0]], rows, sem)
        fill_vals(buf, base)

    def process(buf):
        ridx, cidx, vals, rows, sem = buf
        pltpu.make_async_copy(table_hbm.at[cidx.at[0]], rows, sem).wait()

        @plsc.parallel_loop(0, K, unroll=8)
        def _(i):
            vi = plsc.load_gather(vals, [lax.broadcast(i, (16,))])
            for f in range(NF):
                sl = pl.ds(f * 16, 16)
                rows[i, sl] = rows[i, sl] * vi

        pltpu.sync_copy(rows, acc.at[ridx.at[0]], add=True)

    load_idx(bufa, 0)

    @pl.loop(0, 2 * ((nchunks - 1) // 2), step=2)
    def _(t):
        load_idx(bufb, t + 1)
        process(bufa)
        load_idx(bufa, t + 2)
        process(bufb)

    if nchunks % 2 == 0:
        load_idx(bufb, nchunks - 1)
        process(bufa)
        process(bufb)
    else:
        process(bufa)


def _edge_buffers():
    return 2 * [
        pltpu.VMEM((1, K), jnp.int32),
        pltpu.VMEM((1, K), jnp.int32),
        pltpu.VMEM((K,), jnp.float32),
        pltpu.VMEM((K, D), jnp.float32),
        pltpu.SemaphoreType.DMA,
    ]


_SC_SCRATCH = [pltpu.VMEM_SHARED((NP, D), jnp.float32)] + _edge_buffers()


@functools.partial(
    pl.kernel,
    out_type=(jax.ShapeDtypeStruct((NP, D), jnp.float32),
              jax.ShapeDtypeStruct((NP, D), jnp.float32)),
    mesh=_MESH,
    compiler_params=_CP,
    scratch_types=_SC_SCRATCH + [
        pltpu.VMEM_SHARED((NP // D, D), jnp.float32),
        pltpu.VMEM((N,), jnp.int32),
        pltpu.VMEM((NP // D, D), jnp.float32),
        pltpu.VMEM((1, NP // D), jnp.int32),
        pltpu.VMEM((32, D), jnp.float32),
        pltpu.VMEM((1, D), jnp.float32),
        pltpu.VMEM((32,), jnp.float32),
    ],
)
def _sc1(t1_hbm, ta_hbm, r1_hbm, c1_hbm, a1_hbm, r2_hbm, c2_hbm, a2_hbm,
         sp_hbm, z1_hbm, z2_hbm,
         acc, ra, ca, va, rwa, sma, rb, cb, vb, rwb, smb,
         den_acc, s_v, den_v, idn, dbuf, drow, dscale):
    core = lax.axis_index("c")
    sid = lax.axis_index("s")
    per_tile = E // NTILE
    nchunks = per_tile // K
    e0 = sid * per_tile
    zero = jnp.zeros((16,), jnp.float32)
    lane = lax.iota(jnp.int32, 16)
    bufa = (ra, ca, va, rwa, sma)
    bufb = (rb, cb, vb, rwb, smb)

    _zero_rows(rwa)
    _zero_acc(acc, rwa, sid)

    @pl.when(jnp.logical_and(core == 1, sid == 0))
    def _():
        pltpu.sync_copy(rwa, den_acc)

    plsc.subcore_barrier()

    @pl.when(core == 0)
    def _():
        def fill_vals(buf, base):
            pltpu.sync_copy(a1_hbm.at[pl.ds(base, K)], buf[2])

        _spmm_accumulate(t1_hbm, r1_hbm, c1_hbm, acc, bufa, bufb,
                         e0, nchunks, fill_vals)

    @pl.when(core == 1)
    def _():
        pltpu.sync_copy(sp_hbm, s_v)

        @pl.loop(0, NP // D)
        def _(i):
            for f in range(NF):
                den_v[i, pl.ds(f * 16, 16)] = zero

        @pl.loop(0, NP // D, step=16)
        def _(j):
            idn[0, pl.ds(j, 16)] = lane + j

        masks = [lane == q for q in range(16)]

        def fill_vals(buf, base):
            ridx, cidx, vals = buf[0], buf[1], buf[2]
            pltpu.sync_copy(a2_hbm.at[pl.ds(base, K)], vals)

            @plsc.parallel_loop(0, K, step=16, unroll=2)
            def _(j):
                sl = pl.ds(j, 16)
                ri = ridx[0, sl]
                p1 = plsc.load_gather(s_v, [ri])
                p2 = plsc.load_gather(s_v, [cidx[0, sl]])
                g1 = lax.shift_right_logical(p1, 16).astype(jnp.float32)
                g2 = (p2 & 0xFFFF).astype(jnp.float32)
                e = vals[sl] * ((g1 + g2 - 65536.0) * (1.0 / 32767.0))
                e = jnp.maximum(e, e * 0.2)
                ee = jnp.exp(e)
                vals[sl] = ee
                rhi = lax.shift_right_logical(ri, 7)
                rlo = ri & (D - 1)
                for q in range(16):
                    plsc.addupdate_scatter(den_v, [rhi, rlo], ee,
                                           mask=masks[q])

        _spmm_accumulate(ta_hbm, r2_hbm, c2_hbm, acc, bufa, bufb,
                         e0, nchunks, fill_vals)

        pltpu.sync_copy(den_v, den_acc.at[idn.at[0]], add=True)

    plsc.subcore_barrier()

    @pl.when(core == 0)
    def _():
        _dump_acc(acc, z1_hbm, sid)

    @pl.when(core == 1)
    def _():
        @pl.loop(sid, NP // 32, step=NTILE)
        def _(m):
            pltpu.sync_copy(acc.at[pl.ds(m * 32, 32)], dbuf)
            pltpu.sync_copy(den_acc.at[pl.ds(m // 4, 1)], drow)
            q = (m % 4) * 32
            for f in range(2):
                dv = drow[0, pl.ds(q + f * 16, 16)]
                dscale[pl.ds(f * 16, 16)] = jnp.where(dv > 0.0, 1.0 / dv, 0.0)

            @plsc.parallel_loop(0, 32, unroll=4)
            def _(i):
                si = plsc.load_gather(dscale, [lax.broadcast(i, (16,))])
                for f in range(NF):
                    sl = pl.ds(f * 16, 16)
                    dbuf[i, sl] = dbuf[i, sl] * si

            pltpu.sync_copy(dbuf, z2_hbm.at[pl.ds(m * 32, 32)])


@functools.partial(
    pl.kernel,
    out_type=(jax.ShapeDtypeStruct((NP, D), jnp.float32),
              jax.ShapeDtypeStruct((NP, D), jnp.float32)),
    mesh=_MESH,
    compiler_params=_CP,
    scratch_types=_SC_SCRATCH,
)
def _sc2(t2_hbm, r1_hbm, c1_hbm, a1_hbm, za_hbm, zb_hbm,
         acc, ra, ca, va, rwa, sma, rb, cb, vb, rwb, smb):
    core = lax.axis_index("c")
    sid = lax.axis_index("s")
    half = E // 2
    per_tile = half // NTILE
    nchunks = per_tile // K
    e0 = core * half + sid * per_tile

    _zero_rows(rwa)
    _zero_acc(acc, rwa, sid)
    plsc.subcore_barrier()

    def fill_vals(buf, base):
        pltpu.sync_copy(a1_hbm.at[pl.ds(base, K)], buf[2])

    _spmm_accumulate(t2_hbm, r1_hbm, c1_hbm, acc, (ra, ca, va, rwa, sma),
                     (rb, cb, vb, rwb, smb), e0, nchunks, fill_vals)

    plsc.subcore_barrier()

    @pl.when(core == 0)
    def _():
        _dump_acc(acc, za_hbm, sid)

    @pl.when(core == 1)
    def _():
        _dump_acc(acc, zb_hbm, sid)


BT = 2000


def _tc1_body(x_ref, w1_ref, wa_ref, m1_ref, m2_ref,
              t1_ref, ta_ref, sp_ref):
    xb = x_ref[...] * BN_SCALE
    t1_ref[...] = jnp.dot(xb, w1_ref[...], precision=HIGH)
    ta_ref[...] = jnp.dot(xb, wa_ref[...], precision=HIGH)
    s1 = jnp.tanh(jnp.sum(
        jnp.dot(xb, m1_ref[...], precision=HIGH) * xb, axis=1, keepdims=True))
    s2 = jnp.tanh(jnp.sum(
        jnp.dot(xb, m2_ref[...], precision=HIGH) * xb, axis=1, keepdims=True))
    q1 = jnp.round(s1 * 32767.0).astype(jnp.int32) + 32768
    q2 = jnp.round(s2 * 32767.0).astype(jnp.int32) + 32768
    sp_ref[...] = lax.shift_left(q1, 16) | q2


def _tc2_body(z1_ref, y2_ref, b1_ref, wh_ref, w2_ref, y3_ref, t2_ref):
    y1 = jnp.tanh(z1_ref[...] + b1_ref[...])
    i1 = y2_ref[...] * BN_SCALE
    i2 = y1 * BN_SCALE
    gate = jnp.maximum(jnp.tanh(jnp.dot(i1, wh_ref[...], precision=HIGH)), 0.0)
    y3 = jnp.tanh(i2 * (1.0 - gate) + i1 * gate)
    y3_ref[...] = y3
    t2_ref[...] = jnp.dot(y3 * BN_SCALE, w2_ref[...], precision=HIGH)


def _l2n(v):
    return v * lax.rsqrt(jnp.maximum(jnp.sum(v * v, axis=1, keepdims=True),
                                     1e-12))


def _tc3_body(za_ref, zb_ref, b2_ref, y3_ref, x_ref, out_ref):
    y4 = jnp.tanh(za_ref[...] + zb_ref[...] + b2_ref[...])
    y = jnp.concatenate([_l2n(y3_ref[...]), _l2n(y4), _l2n(x_ref[...])],
                        axis=1)
    out_ref[...] = _l2n(y)


def _row_spec(width):
    return pl.BlockSpec((BT, width), lambda i: (i, 0))


def _full_spec(shape):
    return pl.BlockSpec(shape, lambda i: tuple(0 for _ in shape))


_tc1 = pl.pallas_call(
    _tc1_body,
    grid=(N // BT,),
    in_specs=[_row_spec(D)] + [_full_spec((D, D))] * 4,
    out_specs=[_row_spec(D), _row_spec(D), _row_spec(1)],
    out_shape=(jax.ShapeDtypeStruct((N, D), jnp.float32),
               jax.ShapeDtypeStruct((N, D), jnp.float32),
               jax.ShapeDtypeStruct((N, 1), jnp.int32)),
)

_tc2 = pl.pallas_call(
    _tc2_body,
    grid=(N // BT,),
    in_specs=[_row_spec(D), _row_spec(D), _full_spec((1, D)),
              _full_spec((D, D)), _full_spec((D, D))],
    out_specs=[_row_spec(D), _row_spec(D)],
    out_shape=(jax.ShapeDtypeStruct((N, D), jnp.float32),
               jax.ShapeDtypeStruct((N, D), jnp.float32)),
)

_tc3 = pl.pallas_call(
    _tc3_body,
    grid=(N // BT,),
    in_specs=[_row_spec(D), _row_spec(D), _full_spec((1, D)),
              _row_spec(D), _row_spec(D)],
    out_specs=_row_spec(3 * D),
    out_shape=jax.ShapeDtypeStruct((N, 3 * D), jnp.float32),
)


def kernel(init_embeds, W1, b1, Wa, M1, M2, Wh, W2, b2,
           a1_values, a2_values, edge_index1, edge_index2):
    x = init_embeds
    r1, c1 = edge_index1[0], edge_index1[1]
    r2, c2 = edge_index2[0], edge_index2[1]

    t1x, tax, sp = _tc1(x, W1, Wa, M1, M2)
    z1, y2 = _sc1(t1x, tax, r1, c1, a1_values, r2, c2, a2_values,
                  sp.reshape(N))
    y3, t2x = _tc2(z1[:N], y2[:N], b1.reshape(1, D), Wh, W2)
    za, zb = _sc2(t2x, r1, c1, a1_values)
    return _tc3(za[:N], zb[:N], b2.reshape(1, D), y3, x)


# speedup vs baseline: 15.7612x; 1.2456x over previous
import dataclasses
import functools
import math

import jax
import jax.numpy as jnp
from jax import lax
from jax.experimental import pallas as pl
from jax.experimental.pallas import tpu as pltpu
from jax.experimental.pallas import tpu_sc as plsc

N = 10000
NP = 10240
D = 128
E = 320000
K = 80
NTILE = 16
NF = D // 16
BN_SCALE = 1.0 / math.sqrt(1.0 + 1e-3)
HIGH = lax.Precision.HIGHEST

_MESH = plsc.VectorSubcoreMesh(core_axis_name="c", subcore_axis_name="s")
_CP = pltpu.CompilerParams()
if "needs_layout_passes" in pltpu.CompilerParams.__dataclass_fields__:
    _CP = dataclasses.replace(_CP, needs_layout_passes=False)


def _zero_rows(rows):
    zero = jnp.zeros((16,), jnp.float32)

    @pl.loop(0, K)
    def _(i):
        for f in range(NF):
            rows[i, pl.ds(f * 16, 16)] = zero


def _zero_acc(acc, rows, sid):
    @pl.loop(sid, NP // K, step=NTILE)
    def _(t):
        pltpu.sync_copy(rows, acc.at[pl.ds(t * K, K)])


def _dump_acc(acc, out_hbm, sid):
    @pl.loop(sid, NP // K, step=NTILE)
    def _(t):
        pltpu.sync_copy(acc.at[pl.ds(t * K, K)], out_hbm.at[pl.ds(t * K, K)])


def _spmm_accumulate(table_hbm, r_hbm, c_hbm, acc, bufa, bufb,
                     e0, nchunks, fill_vals):
    def load_idx(buf, t):
        ridx, cidx, vals, rows, sem = buf
        base = e0 + t * K
        cr = pltpu.async_copy(r_hbm.at[pl.ds(base, K)], ridx.at[0], sem)
        cc = pltpu.async_copy(c_hbm.at[pl.ds(base, K)], cidx.at[0], sem)
        cr.wait()
        cc.wait()
        pltpu.async_copy(table_hbm.at[cidx.at[0]], rows, sem)
        fill_vals(buf, base)

    def scale(buf):
        ridx, cidx, vals, rows, sem = buf
        pltpu.make_async_copy(table_hbm.at[cidx.at[0]], rows, sem).wait()

        @plsc.parallel_loop(0, K, unroll=8)
        def _(i):
            vi = plsc.load_gather(vals, [lax.broadcast(i, (16,))])
            for f in range(NF):
                sl = pl.ds(f * 16, 16)
                rows[i, sl] = rows[i, sl] * vi

    def scatter_start(buf):
        ridx, cidx, vals, rows, sem = buf
        pltpu.async_copy(rows, acc.at[ridx.at[0]], sem, add=True)

    def scatter_wait(buf):
        ridx, cidx, vals, rows, sem = buf
        pltpu.make_async_copy(rows, acc.at[ridx.at[0]], sem).wait()

    load_idx(bufa, 0)
    load_idx(bufb, 1)
    scale(bufa)
    scatter_start(bufa)
    scale(bufb)
    scatter_wait(bufa)
    load_idx(bufa, 2)
    scatter_start(bufb)

    @pl.loop(2, 2 * ((nchunks - 1) // 2), step=2)
    def _(t):
        scatter_wait(bufb)
        load_idx(bufb, t + 1)
        scale(bufa)
        scatter_start(bufa)
        scale(bufb)
        scatter_wait(bufa)
        load_idx(bufa, t + 2)
        scatter_start(bufb)

    if nchunks % 2 == 0:
        scatter_wait(bufb)
        load_idx(bufb, nchunks - 1)
        scale(bufa)
        scatter_start(bufa)
        scale(bufb)
        scatter_wait(bufa)
        scatter_start(bufb)
        scatter_wait(bufb)
    else:
        scale(bufa)
        scatter_wait(bufb)
        scatter_start(bufa)
        scatter_wait(bufa)


def _edge_buffers():
    return 2 * [
        pltpu.VMEM((1, K), jnp.int32),
        pltpu.VMEM((1, K), jnp.int32),
        pltpu.VMEM((K,), jnp.float32),
        pltpu.VMEM((K, D), jnp.float32),
        pltpu.SemaphoreType.DMA,
    ]


_SC_SCRATCH = [pltpu.VMEM_SHARED((NP, D), jnp.float32)] + _edge_buffers()


@functools.partial(
    pl.kernel,
    out_type=(jax.ShapeDtypeStruct((NP, D), jnp.float32),
              jax.ShapeDtypeStruct((NP, D), jnp.float32)),
    mesh=_MESH,
    compiler_params=_CP,
    scratch_types=_SC_SCRATCH + [
        pltpu.VMEM_SHARED((NP // D, D), jnp.float32),
        pltpu.VMEM((N,), jnp.int32),
        pltpu.VMEM((NP // D, D), jnp.float32),
        pltpu.VMEM((1, NP // D), jnp.int32),
        pltpu.VMEM((32, D), jnp.float32),
        pltpu.VMEM((1, D), jnp.float32),
        pltpu.VMEM((32,), jnp.float32),
    ],
)
def _sc1(t1_hbm, ta_hbm, r1_hbm, c1_hbm, a1_hbm, r2_hbm, c2_hbm, a2_hbm,
         sp_hbm, z1_hbm, z2_hbm,
         acc, ra, ca, va, rwa, sma, rb, cb, vb, rwb, smb,
         den_acc, s_v, den_v, idn, dbuf, drow, dscale):
    core = lax.axis_index("c")
    sid = lax.axis_index("s")
    per_tile = E // NTILE
    nchunks = per_tile // K
    e0 = sid * per_tile
    zero = jnp.zeros((16,), jnp.float32)
    lane = lax.iota(jnp.int32, 16)
    bufa = (ra, ca, va, rwa, sma)
    bufb = (rb, cb, vb, rwb, smb)

    _zero_rows(rwa)
    _zero_acc(acc, rwa, sid)

    @pl.when(jnp.logical_and(core == 1, sid == 0))
    def _():
        pltpu.sync_copy(rwa, den_acc)

    plsc.subcore_barrier()

    @pl.when(core == 0)
    def _():
        def fill_vals(buf, base):
            pltpu.sync_copy(a1_hbm.at[pl.ds(base, K)], buf[2])

        _spmm_accumulate(t1_hbm, r1_hbm, c1_hbm, acc, bufa, bufb,
                         e0, nchunks, fill_vals)

    @pl.when(core == 1)
    def _():
        pltpu.sync_copy(sp_hbm, s_v)

        @pl.loop(0, NP // D)
        def _(i):
            for f in range(NF):
                den_v[i, pl.ds(f * 16, 16)] = zero

        @pl.loop(0, NP // D, step=16)
        def _(j):
            idn[0, pl.ds(j, 16)] = lane + j

        masks = [lane == q for q in range(16)]

        def fill_vals(buf, base):
            ridx, cidx, vals = buf[0], buf[1], buf[2]
            pltpu.sync_copy(a2_hbm.at[pl.ds(base, K)], vals)

            @plsc.parallel_loop(0, K, step=16, unroll=2)
            def _(j):
                sl = pl.ds(j, 16)
                ri = ridx[0, sl]
                p1 = plsc.load_gather(s_v, [ri])
                p2 = plsc.load_gather(s_v, [cidx[0, sl]])
                g1 = lax.shift_right_logical(p1, 16).astype(jnp.float32)
                g2 = (p2 & 0xFFFF).astype(jnp.float32)
                e = vals[sl] * ((g1 + g2 - 65536.0) * (1.0 / 32767.0))
                e = jnp.maximum(e, e * 0.2)
                ee = jnp.exp(e)
                vals[sl] = ee
                rhi = lax.shift_right_logical(ri, 7)
                rlo = ri & (D - 1)
                for q in range(16):
                    plsc.addupdate_scatter(den_v, [rhi, rlo], ee,
                                           mask=masks[q])

        _spmm_accumulate(ta_hbm, r2_hbm, c2_hbm, acc, bufa, bufb,
                         e0, nchunks, fill_vals)

        pltpu.sync_copy(den_v, den_acc.at[idn.at[0]], add=True)

    plsc.subcore_barrier()

    @pl.when(core == 0)
    def _():
        _dump_acc(acc, z1_hbm, sid)

    @pl.when(core == 1)
    def _():
        @pl.loop(sid, NP // 32, step=NTILE)
        def _(m):
            pltpu.sync_copy(acc.at[pl.ds(m * 32, 32)], dbuf)
            pltpu.sync_copy(den_acc.at[pl.ds(m // 4, 1)], drow)
            q = (m % 4) * 32
            for f in range(2):
                dv = drow[0, pl.ds(q + f * 16, 16)]
                dscale[pl.ds(f * 16, 16)] = jnp.where(dv > 0.0, 1.0 / dv, 0.0)

            @plsc.parallel_loop(0, 32, unroll=4)
            def _(i):
                si = plsc.load_gather(dscale, [lax.broadcast(i, (16,))])
                for f in range(NF):
                    sl = pl.ds(f * 16, 16)
                    dbuf[i, sl] = dbuf[i, sl] * si

            pltpu.sync_copy(dbuf, z2_hbm.at[pl.ds(m * 32, 32)])


@functools.partial(
    pl.kernel,
    out_type=(jax.ShapeDtypeStruct((NP, D), jnp.float32),
              jax.ShapeDtypeStruct((NP, D), jnp.float32)),
    mesh=_MESH,
    compiler_params=_CP,
    scratch_types=_SC_SCRATCH,
)
def _sc2(t2_hbm, r1_hbm, c1_hbm, a1_hbm, za_hbm, zb_hbm,
         acc, ra, ca, va, rwa, sma, rb, cb, vb, rwb, smb):
    core = lax.axis_index("c")
    sid = lax.axis_index("s")
    half = E // 2
    per_tile = half // NTILE
    nchunks = per_tile // K
    e0 = core * half + sid * per_tile

    _zero_rows(rwa)
    _zero_acc(acc, rwa, sid)
    plsc.subcore_barrier()

    def fill_vals(buf, base):
        pltpu.sync_copy(a1_hbm.at[pl.ds(base, K)], buf[2])

    _spmm_accumulate(t2_hbm, r1_hbm, c1_hbm, acc, (ra, ca, va, rwa, sma),
                     (rb, cb, vb, rwb, smb), e0, nchunks, fill_vals)

    plsc.subcore_barrier()

    @pl.when(core == 0)
    def _():
        _dump_acc(acc, za_hbm, sid)

    @pl.when(core == 1)
    def _():
        _dump_acc(acc, zb_hbm, sid)


BT = 2000


def _tc1_body(x_ref, w1_ref, wa_ref, m1_ref, m2_ref,
              t1_ref, ta_ref, sp_ref):
    xb = x_ref[...] * BN_SCALE
    t1_ref[...] = jnp.dot(xb, w1_ref[...], precision=HIGH)
    ta_ref[...] = jnp.dot(xb, wa_ref[...], precision=HIGH)
    s1 = jnp.tanh(jnp.sum(
        jnp.dot(xb, m1_ref[...], precision=HIGH) * xb, axis=1, keepdims=True))
    s2 = jnp.tanh(jnp.sum(
        jnp.dot(xb, m2_ref[...], precision=HIGH) * xb, axis=1, keepdims=True))
    q1 = jnp.round(s1 * 32767.0).astype(jnp.int32) + 32768
    q2 = jnp.round(s2 * 32767.0).astype(jnp.int32) + 32768
    sp_ref[...] = lax.shift_left(q1, 16) | q2


def _tc2_body(z1_ref, y2_ref, b1_ref, wh_ref, w2_ref, y3_ref, t2_ref):
    y1 = jnp.tanh(z1_ref[...] + b1_ref[...])
    i1 = y2_ref[...] * BN_SCALE
    i2 = y1 * BN_SCALE
    gate = jnp.maximum(jnp.tanh(jnp.dot(i1, wh_ref[...], precision=HIGH)), 0.0)
    y3 = jnp.tanh(i2 * (1.0 - gate) + i1 * gate)
    y3_ref[...] = y3
    t2_ref[...] = jnp.dot(y3 * BN_SCALE, w2_ref[...], precision=HIGH)


def _l2n(v):
    return v * lax.rsqrt(jnp.maximum(jnp.sum(v * v, axis=1, keepdims=True),
                                     1e-12))


def _tc3_body(za_ref, zb_ref, b2_ref, y3_ref, x_ref, out_ref):
    y4 = jnp.tanh(za_ref[...] + zb_ref[...] + b2_ref[...])
    y = jnp.concatenate([_l2n(y3_ref[...]), _l2n(y4), _l2n(x_ref[...])],
                        axis=1)
    out_ref[...] = _l2n(y)


def _row_spec(width):
    return pl.BlockSpec((BT, width), lambda i: (i, 0))


def _full_spec(shape):
    return pl.BlockSpec(shape, lambda i: tuple(0 for _ in shape))


_tc1 = pl.pallas_call(
    _tc1_body,
    grid=(N // BT,),
    in_specs=[_row_spec(D)] + [_full_spec((D, D))] * 4,
    out_specs=[_row_spec(D), _row_spec(D), _row_spec(1)],
    out_shape=(jax.ShapeDtypeStruct((N, D), jnp.float32),
               jax.ShapeDtypeStruct((N, D), jnp.float32),
               jax.ShapeDtypeStruct((N, 1), jnp.int32)),
)

_tc2 = pl.pallas_call(
    _tc2_body,
    grid=(N // BT,),
    in_specs=[_row_spec(D), _row_spec(D), _full_spec((1, D)),
              _full_spec((D, D)), _full_spec((D, D))],
    out_specs=[_row_spec(D), _row_spec(D)],
    out_shape=(jax.ShapeDtypeStruct((N, D), jnp.float32),
               jax.ShapeDtypeStruct((N, D), jnp.float32)),
)

_tc3 = pl.pallas_call(
    _tc3_body,
    grid=(N // BT,),
    in_specs=[_row_spec(D), _row_spec(D), _full_spec((1, D)),
              _row_spec(D), _row_spec(D)],
    out_specs=_row_spec(3 * D),
    out_shape=jax.ShapeDtypeStruct((N, 3 * D), jnp.float32),
)


def kernel(init_embeds, W1, b1, Wa, M1, M2, Wh, W2, b2,
           a1_values, a2_values, edge_index1, edge_index2):
    x = init_embeds
    r1, c1 = edge_index1[0], edge_index1[1]
    r2, c2 = edge_index2[0], edge_index2[1]

    t1x, tax, sp = _tc1(x, W1, Wa, M1, M2)
    z1, y2 = _sc1(t1x, tax, r1, c1, a1_values, r2, c2, a2_values,
                  sp.reshape(N))
    y3, t2x = _tc2(z1[:N], y2[:N], b1.reshape(1, D), Wh, W2)
    za, zb = _sc2(t2x, r1, c1, a1_values)
    return _tc3(za[:N], zb[:N], b2.reshape(1, D), y3, x)


# speedup vs baseline: 16.2693x; 1.0322x over previous
import dataclasses
import functools
import math

import jax
import jax.numpy as jnp
from jax import lax
from jax.experimental import pallas as pl
from jax.experimental.pallas import tpu as pltpu
from jax.experimental.pallas import tpu_sc as plsc

N = 10000
NP = 10240
D = 128
E = 320000
K = 80
NTILE = 16
NF = D // 16
BN_SCALE = 1.0 / math.sqrt(1.0 + 1e-3)
HIGH = lax.Precision.HIGHEST

_MESH = plsc.VectorSubcoreMesh(core_axis_name="c", subcore_axis_name="s")
_CP = pltpu.CompilerParams()
if "needs_layout_passes" in pltpu.CompilerParams.__dataclass_fields__:
    _CP = dataclasses.replace(_CP, needs_layout_passes=False)


def _zero_rows(rows):
    zero = jnp.zeros((16,), jnp.float32)

    @pl.loop(0, K)
    def _(i):
        for f in range(NF):
            rows[i, pl.ds(f * 16, 16)] = zero


def _zero_acc(acc, rows, sid):
    @pl.loop(sid, NP // K, step=NTILE)
    def _(t):
        pltpu.sync_copy(rows, acc.at[pl.ds(t * K, K)])


def _dump_acc(acc, out_hbm, sid):
    @pl.loop(sid, NP // K, step=NTILE)
    def _(t):
        pltpu.sync_copy(acc.at[pl.ds(t * K, K)], out_hbm.at[pl.ds(t * K, K)])


def _spmm_accumulate(table_hbm, r_hbm, c_hbm, acc, bufa, bufb,
                     e0, nchunks, fill_vals, deferred_vals=False):
    def load_idx(buf, t):
        ridx, cidx, vals, rows, sem, vsem = buf
        base = e0 + t * K
        cr = pltpu.async_copy(r_hbm.at[pl.ds(base, K)], ridx.at[0], sem)
        cc = pltpu.async_copy(c_hbm.at[pl.ds(base, K)], cidx.at[0], sem)
        cr.wait()
        cc.wait()
        pltpu.async_copy(table_hbm.at[cidx.at[0]], rows, sem)
        fill_vals(buf, base)

    def scale(buf):
        ridx, cidx, vals, rows, sem, vsem = buf
        pltpu.make_async_copy(table_hbm.at[cidx.at[0]], rows, sem).wait()
        if deferred_vals:
            pltpu.make_async_copy(r_hbm.at[pl.ds(0, K)], vals, vsem).wait()

        @plsc.parallel_loop(0, K, unroll=8)
        def _(i):
            vi = plsc.load_gather(vals, [lax.broadcast(i, (16,))])
            for f in range(NF):
                sl = pl.ds(f * 16, 16)
                rows[i, sl] = rows[i, sl] * vi

    def scatter_start(buf):
        ridx, cidx, vals, rows, sem, vsem = buf
        pltpu.async_copy(rows, acc.at[ridx.at[0]], sem, add=True)

    def scatter_wait(buf):
        ridx, cidx, vals, rows, sem, vsem = buf
        pltpu.make_async_copy(rows, acc.at[ridx.at[0]], sem).wait()

    load_idx(bufa, 0)
    load_idx(bufb, 1)
    scale(bufa)
    scatter_start(bufa)
    scale(bufb)
    scatter_wait(bufa)
    load_idx(bufa, 2)
    scatter_start(bufb)

    @pl.loop(2, 2 * ((nchunks - 1) // 2), step=2)
    def _(t):
        scatter_wait(bufb)
        load_idx(bufb, t + 1)
        scale(bufa)
        scatter_start(bufa)
        scale(bufb)
        scatter_wait(bufa)
        load_idx(bufa, t + 2)
        scatter_start(bufb)

    if nchunks % 2 == 0:
        scatter_wait(bufb)
        load_idx(bufb, nchunks - 1)
        scale(bufa)
        scatter_start(bufa)
        scale(bufb)
        scatter_wait(bufa)
        scatter_start(bufb)
        scatter_wait(bufb)
    else:
        scale(bufa)
        scatter_wait(bufb)
        scatter_start(bufa)
        scatter_wait(bufa)


def _edge_buffers():
    return 2 * [
        pltpu.VMEM((1, K), jnp.int32),
        pltpu.VMEM((1, K), jnp.int32),
        pltpu.VMEM((K,), jnp.float32),
        pltpu.VMEM((K, D), jnp.float32),
        pltpu.SemaphoreType.DMA,
        pltpu.SemaphoreType.DMA,
    ]


_SC_SCRATCH = [pltpu.VMEM_SHARED((NP, D), jnp.float32)] + _edge_buffers()


@functools.partial(
    pl.kernel,
    out_type=(jax.ShapeDtypeStruct((NP, D), jnp.float32),
              jax.ShapeDtypeStruct((NP, D), jnp.float32)),
    mesh=_MESH,
    compiler_params=_CP,
    scratch_types=_SC_SCRATCH + [
        pltpu.VMEM_SHARED((NP // D, D), jnp.float32),
        pltpu.VMEM((N,), jnp.int32),
        pltpu.VMEM((NP // D, D), jnp.float32),
        pltpu.VMEM((1, NP // D), jnp.int32),
        pltpu.VMEM((32, D), jnp.float32),
        pltpu.VMEM((1, D), jnp.float32),
        pltpu.VMEM((32,), jnp.float32),
    ],
)
def _sc1(t1_hbm, ta_hbm, r1_hbm, c1_hbm, a1_hbm, r2_hbm, c2_hbm, a2_hbm,
         sp_hbm, z1_hbm, z2_hbm,
         acc, ra, ca, va, rwa, sma, vsma, rb, cb, vb, rwb, smb, vsmb,
         den_acc, s_v, den_v, idn, dbuf, drow, dscale):
    core = lax.axis_index("c")
    sid = lax.axis_index("s")
    per_tile = E // NTILE
    nchunks = per_tile // K
    e0 = sid * per_tile
    zero = jnp.zeros((16,), jnp.float32)
    lane = lax.iota(jnp.int32, 16)
    bufa = (ra, ca, va, rwa, sma, vsma)
    bufb = (rb, cb, vb, rwb, smb, vsmb)

    _zero_rows(rwa)
    _zero_acc(acc, rwa, sid)

    @pl.when(jnp.logical_and(core == 1, sid == 0))
    def _():
        pltpu.sync_copy(rwa, den_acc)

    plsc.subcore_barrier()

    @pl.when(core == 0)
    def _():
        def fill_vals(buf, base):
            pltpu.async_copy(a1_hbm.at[pl.ds(base, K)], buf[2], buf[5])

        _spmm_accumulate(t1_hbm, r1_hbm, c1_hbm, acc, bufa, bufb,
                         e0, nchunks, fill_vals, deferred_vals=True)

    @pl.when(core == 1)
    def _():
        pltpu.sync_copy(sp_hbm, s_v)

        @pl.loop(0, NP // D)
        def _(i):
            for f in range(NF):
                den_v[i, pl.ds(f * 16, 16)] = zero

        @pl.loop(0, NP // D, step=16)
        def _(j):
            idn[0, pl.ds(j, 16)] = lane + j

        masks = [lane == q for q in range(16)]

        def fill_vals(buf, base):
            ridx, cidx, vals = buf[0], buf[1], buf[2]
            pltpu.sync_copy(a2_hbm.at[pl.ds(base, K)], vals)

            @plsc.parallel_loop(0, K, step=16, unroll=2)
            def _(j):
                sl = pl.ds(j, 16)
                ri = ridx[0, sl]
                p1 = plsc.load_gather(s_v, [ri])
                p2 = plsc.load_gather(s_v, [cidx[0, sl]])
                g1 = lax.shift_right_logical(p1, 16).astype(jnp.float32)
                g2 = (p2 & 0xFFFF).astype(jnp.float32)
                e = vals[sl] * ((g1 + g2 - 65536.0) * (1.0 / 32767.0))
                e = jnp.maximum(e, e * 0.2)
                ee = jnp.exp(e)
                vals[sl] = ee
                rhi = lax.shift_right_logical(ri, 7)
                rlo = ri & (D - 1)
                for q in range(16):
                    plsc.addupdate_scatter(den_v, [rhi, rlo], ee,
                                           mask=masks[q])

        _spmm_accumulate(ta_hbm, r2_hbm, c2_hbm, acc, bufa, bufb,
                         e0, nchunks, fill_vals)

        pltpu.sync_copy(den_v, den_acc.at[idn.at[0]], add=True)

    plsc.subcore_barrier()

    @pl.when(core == 0)
    def _():
        _dump_acc(acc, z1_hbm, sid)

    @pl.when(core == 1)
    def _():
        @pl.loop(sid, NP // 32, step=NTILE)
        def _(m):
            pltpu.sync_copy(acc.at[pl.ds(m * 32, 32)], dbuf)
            pltpu.sync_copy(den_acc.at[pl.ds(m // 4, 1)], drow)
            q = (m % 4) * 32
            for f in range(2):
                dv = drow[0, pl.ds(q + f * 16, 16)]
                dscale[pl.ds(f * 16, 16)] = jnp.where(dv > 0.0, 1.0 / dv, 0.0)

            @plsc.parallel_loop(0, 32, unroll=4)
            def _(i):
                si = plsc.load_gather(dscale, [lax.broadcast(i, (16,))])
                for f in range(NF):
                    sl = pl.ds(f * 16, 16)
                    dbuf[i, sl] = dbuf[i, sl] * si

            pltpu.sync_copy(dbuf, z2_hbm.at[pl.ds(m * 32, 32)])


@functools.partial(
    pl.kernel,
    out_type=(jax.ShapeDtypeStruct((NP, D), jnp.float32),
              jax.ShapeDtypeStruct((NP, D), jnp.float32)),
    mesh=_MESH,
    compiler_params=_CP,
    scratch_types=_SC_SCRATCH,
)
def _sc2(t2_hbm, r1_hbm, c1_hbm, a1_hbm, za_hbm, zb_hbm,
         acc, ra, ca, va, rwa, sma, vsma, rb, cb, vb, rwb, smb, vsmb):
    core = lax.axis_index("c")
    sid = lax.axis_index("s")
    half = E // 2
    per_tile = half // NTILE
    nchunks = per_tile // K
    e0 = core * half + sid * per_tile

    _zero_rows(rwa)
    _zero_acc(acc, rwa, sid)
    plsc.subcore_barrier()

    def fill_vals(buf, base):
        pltpu.async_copy(a1_hbm.at[pl.ds(base, K)], buf[2], buf[5])

    _spmm_accumulate(t2_hbm, r1_hbm, c1_hbm, acc,
                     (ra, ca, va, rwa, sma, vsma),
                     (rb, cb, vb, rwb, smb, vsmb),
                     e0, nchunks, fill_vals, deferred_vals=True)

    plsc.subcore_barrier()

    @pl.when(core == 0)
    def _():
        _dump_acc(acc, za_hbm, sid)

    @pl.when(core == 1)
    def _():
        _dump_acc(acc, zb_hbm, sid)


BT = 2000


def _tc1_body(x_ref, w1_ref, wa_ref, m1_ref, m2_ref,
              t1_ref, ta_ref, sp_ref):
    xb = x_ref[...] * BN_SCALE
    t1_ref[...] = jnp.dot(xb, w1_ref[...], precision=HIGH)
    ta_ref[...] = jnp.dot(xb, wa_ref[...], precision=HIGH)
    s1 = jnp.tanh(jnp.sum(
        jnp.dot(xb, m1_ref[...], precision=HIGH) * xb, axis=1, keepdims=True))
    s2 = jnp.tanh(jnp.sum(
        jnp.dot(xb, m2_ref[...], precision=HIGH) * xb, axis=1, keepdims=True))
    q1 = jnp.round(s1 * 32767.0).astype(jnp.int32) + 32768
    q2 = jnp.round(s2 * 32767.0).astype(jnp.int32) + 32768
    sp_ref[...] = lax.shift_left(q1, 16) | q2


def _tc2_body(z1_ref, y2_ref, b1_ref, wh_ref, w2_ref, y3_ref, t2_ref):
    y1 = jnp.tanh(z1_ref[...] + b1_ref[...])
    i1 = y2_ref[...] * BN_SCALE
    i2 = y1 * BN_SCALE
    gate = jnp.maximum(jnp.tanh(jnp.dot(i1, wh_ref[...], precision=HIGH)), 0.0)
    y3 = jnp.tanh(i2 * (1.0 - gate) + i1 * gate)
    y3_ref[...] = y3
    t2_ref[...] = jnp.dot(y3 * BN_SCALE, w2_ref[...], precision=HIGH)


def _l2n(v):
    return v * lax.rsqrt(jnp.maximum(jnp.sum(v * v, axis=1, keepdims=True),
                                     1e-12))


def _tc3_body(za_ref, zb_ref, b2_ref, y3_ref, x_ref, out_ref):
    y4 = jnp.tanh(za_ref[...] + zb_ref[...] + b2_ref[...])
    y = jnp.concatenate([_l2n(y3_ref[...]), _l2n(y4), _l2n(x_ref[...])],
                        axis=1)
    out_ref[...] = _l2n(y)


def _row_spec(width):
    return pl.BlockSpec((BT, width), lambda i: (i, 0))


def _full_spec(shape):
    return pl.BlockSpec(shape, lambda i: tuple(0 for _ in shape))


_tc1 = pl.pallas_call(
    _tc1_body,
    grid=(N // BT,),
    in_specs=[_row_spec(D)] + [_full_spec((D, D))] * 4,
    out_specs=[_row_spec(D), _row_spec(D), _row_spec(1)],
    out_shape=(jax.ShapeDtypeStruct((N, D), jnp.float32),
               jax.ShapeDtypeStruct((N, D), jnp.float32),
               jax.ShapeDtypeStruct((N, 1), jnp.int32)),
)

_tc2 = pl.pallas_call(
    _tc2_body,
    grid=(N // BT,),
    in_specs=[_row_spec(D), _row_spec(D), _full_spec((1, D)),
              _full_spec((D, D)), _full_spec((D, D))],
    out_specs=[_row_spec(D), _row_spec(D)],
    out_shape=(jax.ShapeDtypeStruct((N, D), jnp.float32),
               jax.ShapeDtypeStruct((N, D), jnp.float32)),
)

_tc3 = pl.pallas_call(
    _tc3_body,
    grid=(N // BT,),
    in_specs=[_row_spec(D), _row_spec(D), _full_spec((1, D)),
              _row_spec(D), _row_spec(D)],
    out_specs=_row_spec(3 * D),
    out_shape=jax.ShapeDtypeStruct((N, 3 * D), jnp.float32),
)


def kernel(init_embeds, W1, b1, Wa, M1, M2, Wh, W2, b2,
           a1_values, a2_values, edge_index1, edge_index2):
    x = init_embeds
    r1, c1 = edge_index1[0], edge_index1[1]
    r2, c2 = edge_index2[0], edge_index2[1]

    t1x, tax, sp = _tc1(x, W1, Wa, M1, M2)
    z1, y2 = _sc1(t1x, tax, r1, c1, a1_values, r2, c2, a2_values,
                  sp.reshape(N))
    y3, t2x = _tc2(z1[:N], y2[:N], b1.reshape(1, D), Wh, W2)
    za, zb = _sc2(t2x, r1, c1, a1_values)
    return _tc3(za[:N], zb[:N], b2.reshape(1, D), y3, x)


# speedup vs baseline: 17.4667x; 1.0736x over previous
import dataclasses
import functools
import math

import jax
import jax.numpy as jnp
from jax import lax
from jax.experimental import pallas as pl
from jax.experimental.pallas import tpu as pltpu
from jax.experimental.pallas import tpu_sc as plsc

N = 10000
NP = 10240
D = 128
E = 320000
K = 80
NTILE = 16
NF = D // 16
BN_SCALE = 1.0 / math.sqrt(1.0 + 1e-3)
HIGH = lax.Precision.HIGHEST

_MESH = plsc.VectorSubcoreMesh(core_axis_name="c", subcore_axis_name="s")
_CP = pltpu.CompilerParams()
if "needs_layout_passes" in pltpu.CompilerParams.__dataclass_fields__:
    _CP = dataclasses.replace(_CP, needs_layout_passes=False)


def _zero_rows(rows):
    zero = jnp.zeros((16,), jnp.float32)

    @pl.loop(0, K)
    def _(i):
        for f in range(NF):
            rows[i, pl.ds(f * 16, 16)] = zero


def _zero_acc(acc, rows, sid):
    @pl.loop(sid, NP // K, step=NTILE)
    def _(t):
        pltpu.sync_copy(rows, acc.at[pl.ds(t * K, K)])


def _dump_acc(acc, out_hbm, sid):
    @pl.loop(sid, NP // K, step=NTILE)
    def _(t):
        pltpu.sync_copy(acc.at[pl.ds(t * K, K)], out_hbm.at[pl.ds(t * K, K)])


def _spmm_accumulate(table_hbm, r_hbm, c_hbm, acc, bufa, bufb,
                     e0, nchunks, fill_start, fill_finish=None,
                     deferred_vals=False):
    def load_idx(buf, t):
        ridx, cidx, vals, rows, sem, vsem = buf
        base = e0 + t * K
        cr = pltpu.async_copy(r_hbm.at[pl.ds(base, K)], ridx.at[0], sem)
        cc = pltpu.async_copy(c_hbm.at[pl.ds(base, K)], cidx.at[0], sem)
        fill_start(buf, base)
        cr.wait()
        cc.wait()
        pltpu.async_copy(table_hbm.at[cidx.at[0]], rows, sem)
        if fill_finish is not None:
            fill_finish(buf)

    def scale(buf):
        ridx, cidx, vals, rows, sem, vsem = buf
        pltpu.make_async_copy(table_hbm.at[cidx.at[0]], rows, sem).wait()
        if deferred_vals:
            pltpu.make_async_copy(r_hbm.at[pl.ds(0, K)], vals, vsem).wait()

        @plsc.parallel_loop(0, K, unroll=8)
        def _(i):
            vi = plsc.load_gather(vals, [lax.broadcast(i, (16,))])
            for f in range(NF):
                sl = pl.ds(f * 16, 16)
                rows[i, sl] = rows[i, sl] * vi

    def scatter_start(buf):
        ridx, cidx, vals, rows, sem, vsem = buf
        pltpu.async_copy(rows, acc.at[ridx.at[0]], sem, add=True)

    def scatter_wait(buf):
        ridx, cidx, vals, rows, sem, vsem = buf
        pltpu.make_async_copy(rows, acc.at[ridx.at[0]], sem).wait()

    load_idx(bufa, 0)
    load_idx(bufb, 1)
    scale(bufa)
    scatter_start(bufa)
    scale(bufb)
    scatter_wait(bufa)
    load_idx(bufa, 2)
    scatter_start(bufb)

    @pl.loop(2, 2 * ((nchunks - 1) // 2), step=2)
    def _(t):
        scatter_wait(bufb)
        load_idx(bufb, t + 1)
        scale(bufa)
        scatter_start(bufa)
        scale(bufb)
        scatter_wait(bufa)
        load_idx(bufa, t + 2)
        scatter_start(bufb)

    if nchunks % 2 == 0:
        scatter_wait(bufb)
        load_idx(bufb, nchunks - 1)
        scale(bufa)
        scatter_start(bufa)
        scale(bufb)
        scatter_wait(bufa)
        scatter_start(bufb)
        scatter_wait(bufb)
    else:
        scale(bufa)
        scatter_wait(bufb)
        scatter_start(bufa)
        scatter_wait(bufa)


def _edge_buffers():
    return 2 * [
        pltpu.VMEM((1, K), jnp.int32),
        pltpu.VMEM((1, K), jnp.int32),
        pltpu.VMEM((K,), jnp.float32),
        pltpu.VMEM((K, D), jnp.float32),
        pltpu.SemaphoreType.DMA,
        pltpu.SemaphoreType.DMA,
    ]


_SC_SCRATCH = [pltpu.VMEM_SHARED((NP, D), jnp.float32)] + _edge_buffers()


@functools.partial(
    pl.kernel,
    out_type=(jax.ShapeDtypeStruct((NP, D), jnp.float32),
              jax.ShapeDtypeStruct((NP, D), jnp.float32)),
    mesh=_MESH,
    compiler_params=_CP,
    scratch_types=_SC_SCRATCH + [
        pltpu.VMEM_SHARED((NP // D, D), jnp.float32),
        pltpu.VMEM((N,), jnp.int32),
        pltpu.VMEM((NP // D, D), jnp.float32),
        pltpu.VMEM((1, NP // D), jnp.int32),
        pltpu.VMEM((32, D), jnp.float32),
        pltpu.VMEM((1, D), jnp.float32),
        pltpu.VMEM((32,), jnp.float32),
    ],
)
def _sc1(t1_hbm, ta_hbm, r1_hbm, c1_hbm, a1_hbm, r2_hbm, c2_hbm, a2_hbm,
         sp_hbm, z1_hbm, z2_hbm,
         acc, ra, ca, va, rwa, sma, vsma, rb, cb, vb, rwb, smb, vsmb,
         den_acc, s_v, den_v, idn, dbuf, drow, dscale):
    core = lax.axis_index("c")
    sid = lax.axis_index("s")
    per_tile = E // NTILE
    nchunks = per_tile // K
    e0 = sid * per_tile
    zero = jnp.zeros((16,), jnp.float32)
    lane = lax.iota(jnp.int32, 16)
    bufa = (ra, ca, va, rwa, sma, vsma)
    bufb = (rb, cb, vb, rwb, smb, vsmb)

    _zero_rows(rwa)
    _zero_acc(acc, rwa, sid)

    @pl.when(jnp.logical_and(core == 1, sid == 0))
    def _():
        pltpu.sync_copy(rwa, den_acc)

    plsc.subcore_barrier()

    @pl.when(core == 0)
    def _():
        def fill_vals(buf, base):
            pltpu.async_copy(a1_hbm.at[pl.ds(base, K)], buf[2], buf[5])

        _spmm_accumulate(t1_hbm, r1_hbm, c1_hbm, acc, bufa, bufb,
                         e0, nchunks, fill_vals, deferred_vals=True)

    @pl.when(core == 1)
    def _():
        pltpu.sync_copy(sp_hbm, s_v)

        @pl.loop(0, NP // D)
        def _(i):
            for f in range(NF):
                den_v[i, pl.ds(f * 16, 16)] = zero

        @pl.loop(0, NP // D, step=16)
        def _(j):
            idn[0, pl.ds(j, 16)] = lane + j

        masks = [lane == q for q in range(16)]

        def fill_start(buf, base):
            pltpu.async_copy(a2_hbm.at[pl.ds(base, K)], buf[2], buf[5])

        def fill_finish(buf):
            ridx, cidx, vals, vsem = buf[0], buf[1], buf[2], buf[5]
            pltpu.make_async_copy(a2_hbm.at[pl.ds(0, K)], vals, vsem).wait()

            @plsc.parallel_loop(0, K, step=16, unroll=2)
            def _(j):
                sl = pl.ds(j, 16)
                ri = ridx[0, sl]
                p1 = plsc.load_gather(s_v, [ri])
                p2 = plsc.load_gather(s_v, [cidx[0, sl]])
                g1 = lax.shift_right_logical(p1, 16).astype(jnp.float32)
                g2 = (p2 & 0xFFFF).astype(jnp.float32)
                e = vals[sl] * ((g1 + g2 - 65536.0) * (1.0 / 32767.0))
                e = jnp.maximum(e, e * 0.2)
                ee = jnp.exp(e)
                vals[sl] = ee
                rhi = lax.shift_right_logical(ri, 7)
                rlo = ri & (D - 1)
                for q in range(16):
                    plsc.addupdate_scatter(den_v, [rhi, rlo], ee,
                                           mask=masks[q])

        _spmm_accumulate(ta_hbm, r2_hbm, c2_hbm, acc, bufa, bufb,
                         e0, nchunks, fill_start, fill_finish)

        pltpu.sync_copy(den_v, den_acc.at[idn.at[0]], add=True)

    plsc.subcore_barrier()

    @pl.when(core == 0)
    def _():
        _dump_acc(acc, z1_hbm, sid)

    @pl.when(core == 1)
    def _():
        @pl.loop(sid, NP // 32, step=NTILE)
        def _(m):
            pltpu.sync_copy(acc.at[pl.ds(m * 32, 32)], dbuf)
            pltpu.sync_copy(den_acc.at[pl.ds(m // 4, 1)], drow)
            q = (m % 4) * 32
            for f in range(2):
                dv = drow[0, pl.ds(q + f * 16, 16)]
                dscale[pl.ds(f * 16, 16)] = jnp.where(dv > 0.0, 1.0 / dv, 0.0)

            @plsc.parallel_loop(0, 32, unroll=4)
            def _(i):
                si = plsc.load_gather(dscale, [lax.broadcast(i, (16,))])
                for f in range(NF):
                    sl = pl.ds(f * 16, 16)
                    dbuf[i, sl] = dbuf[i, sl] * si

            pltpu.sync_copy(dbuf, z2_hbm.at[pl.ds(m * 32, 32)])


@functools.partial(
    pl.kernel,
    out_type=(jax.ShapeDtypeStruct((NP, D), jnp.float32),
              jax.ShapeDtypeStruct((NP, D), jnp.float32)),
    mesh=_MESH,
    compiler_params=_CP,
    scratch_types=_SC_SCRATCH,
)
def _sc2(t2_hbm, r1_hbm, c1_hbm, a1_hbm, za_hbm, zb_hbm,
         acc, ra, ca, va, rwa, sma, vsma, rb, cb, vb, rwb, smb, vsmb):
    core = lax.axis_index("c")
    sid = lax.axis_index("s")
    half = E // 2
    per_tile = half // NTILE
    nchunks = per_tile // K
    e0 = core * half + sid * per_tile

    _zero_rows(rwa)
    _zero_acc(acc, rwa, sid)
    plsc.subcore_barrier()

    def fill_vals(buf, base):
        pltpu.async_copy(a1_hbm.at[pl.ds(base, K)], buf[2], buf[5])

    _spmm_accumulate(t2_hbm, r1_hbm, c1_hbm, acc,
                     (ra, ca, va, rwa, sma, vsma),
                     (rb, cb, vb, rwb, smb, vsmb),
                     e0, nchunks, fill_vals, deferred_vals=True)

    plsc.subcore_barrier()

    @pl.when(core == 0)
    def _():
        _dump_acc(acc, za_hbm, sid)

    @pl.when(core == 1)
    def _():
        _dump_acc(acc, zb_hbm, sid)


BT = 2000


def _tc1_body(x_ref, w1_ref, wa_ref, m1_ref, m2_ref,
              t1_ref, ta_ref, sp_ref):
    xb = x_ref[...] * BN_SCALE
    t1_ref[...] = jnp.dot(xb, w1_ref[...], precision=HIGH)
    ta_ref[...] = jnp.dot(xb, wa_ref[...], precision=HIGH)
    s1 = jnp.tanh(jnp.sum(
        jnp.dot(xb, m1_ref[...], precision=HIGH) * xb, axis=1, keepdims=True))
    s2 = jnp.tanh(jnp.sum(
        jnp.dot(xb, m2_ref[...], precision=HIGH) * xb, axis=1, keepdims=True))
    q1 = jnp.round(s1 * 32767.0).astype(jnp.int32) + 32768
    q2 = jnp.round(s2 * 32767.0).astype(jnp.int32) + 32768
    sp_ref[...] = lax.shift_left(q1, 16) | q2


def _tc2_body(z1_ref, y2_ref, b1_ref, wh_ref, w2_ref, y3_ref, t2_ref):
    y1 = jnp.tanh(z1_ref[...] + b1_ref[...])
    i1 = y2_ref[...] * BN_SCALE
    i2 = y1 * BN_SCALE
    gate = jnp.maximum(jnp.tanh(jnp.dot(i1, wh_ref[...], precision=HIGH)), 0.0)
    y3 = jnp.tanh(i2 * (1.0 - gate) + i1 * gate)
    y3_ref[...] = y3
    t2_ref[...] = jnp.dot(y3 * BN_SCALE, w2_ref[...], precision=HIGH)


def _l2n(v):
    return v * lax.rsqrt(jnp.maximum(jnp.sum(v * v, axis=1, keepdims=True),
                                     1e-12))


def _tc3_body(za_ref, zb_ref, b2_ref, y3_ref, x_ref, out_ref):
    y4 = jnp.tanh(za_ref[...] + zb_ref[...] + b2_ref[...])
    y = jnp.concatenate([_l2n(y3_ref[...]), _l2n(y4), _l2n(x_ref[...])],
                        axis=1)
    out_ref[...] = _l2n(y)


def _row_spec(width):
    return pl.BlockSpec((BT, width), lambda i: (i, 0))


def _full_spec(shape):
    return pl.BlockSpec(shape, lambda i: tuple(0 for _ in shape))


_tc1 = pl.pallas_call(
    _tc1_body,
    grid=(N // BT,),
    in_specs=[_row_spec(D)] + [_full_spec((D, D))] * 4,
    out_specs=[_row_spec(D), _row_spec(D), _row_spec(1)],
    out_shape=(jax.ShapeDtypeStruct((N, D), jnp.float32),
               jax.ShapeDtypeStruct((N, D), jnp.float32),
               jax.ShapeDtypeStruct((N, 1), jnp.int32)),
)

_tc2 = pl.pallas_call(
    _tc2_body,
    grid=(N // BT,),
    in_specs=[_row_spec(D), _row_spec(D), _full_spec((1, D)),
              _full_spec((D, D)), _full_spec((D, D))],
    out_specs=[_row_spec(D), _row_spec(D)],
    out_shape=(jax.ShapeDtypeStruct((N, D), jnp.float32),
               jax.ShapeDtypeStruct((N, D), jnp.float32)),
)

_tc3 = pl.pallas_call(
    _tc3_body,
    grid=(N // BT,),
    in_specs=[_row_spec(D), _row_spec(D), _full_spec((1, D)),
              _row_spec(D), _row_spec(D)],
    out_specs=_row_spec(3 * D),
    out_shape=jax.ShapeDtypeStruct((N, 3 * D), jnp.float32),
)


def kernel(init_embeds, W1, b1, Wa, M1, M2, Wh, W2, b2,
           a1_values, a2_values, edge_index1, edge_index2):
    x = init_embeds
    r1, c1 = edge_index1[0], edge_index1[1]
    r2, c2 = edge_index2[0], edge_index2[1]

    t1x, tax, sp = _tc1(x, W1, Wa, M1, M2)
    z1, y2 = _sc1(t1x, tax, r1, c1, a1_values, r2, c2, a2_values,
                  sp.reshape(N))
    y3, t2x = _tc2(z1[:N], y2[:N], b1.reshape(1, D), Wh, W2)
    za, zb = _sc2(t2x, r1, c1, a1_values)
    return _tc3(za[:N], zb[:N], b2.reshape(1, D), y3, x)
